# baseline (speedup 1.0000x reference)
import functools

import jax
import jax.numpy as jnp
import numpy as np
from jax import lax
from jax.experimental import pallas as pl
from jax.experimental.pallas import tpu as pltpu
from jax.experimental.pallas import tpu_sc as plsc

N = 10000
E = 160000
D = 128
ED = 16
NH = 8
DH = 16
NC = 2
NS = 16
NW = NC * NS
CH = 128
NCHUNK = E // CH
RPS = N // NS
INV_SQRT_DH = 1.0 / np.sqrt(DH)

def _mk_mesh():
    return plsc.VectorSubcoreMesh(core_axis_name="c", subcore_axis_name="s",
                                  num_cores=NC, num_subcores=NS)


def _wid():
    return lax.axis_index("s") * NC + lax.axis_index("c")


def _chunk_loop(body, nchunk=NCHUNK):
    w = _wid()
    nch = jnp.where(w < (nchunk % NW), nchunk // NW + 1, nchunk // NW)

    def tbody(t, carry):
        body(t * NW + w)
        return carry

    lax.fori_loop(0, nch, tbody, 0)


def _staged_rows(n, step=64):
    out = []
    o = 0
    while o < n:
        s = min(step, n - o)
        out.append((o, s))
        o += s
    return out


def _sc0_body(pos_hbm, row_hbm, col_hbm, out_hbm,
              idxr0, idxc0, idxr1, idxc1, pa0, pb0, pa1, pb1, ob0, ob1,
              semi0, semi1, semg0, semg1, semw0, semw1):
    w = _wid()
    nch = jnp.where(w < (NCHUNK % NW), NCHUNK // NW + 1, NCHUNK // NW)
    npairs = nch // 2

    def off(t):
        return (t * NW + w) * CH

    def issue_idx(t, idxr, idxc, semi):
        pltpu.async_copy(row_hbm.at[pl.ds(off(t), CH)], idxr, semi)
        pltpu.async_copy(col_hbm.at[pl.ds(off(t), CH)], idxc, semi)

    def wait_idx(idxr, idxc, semi):
        pltpu.make_async_copy(row_hbm.at[pl.ds(0, CH)], idxr, semi).wait()
        pltpu.make_async_copy(col_hbm.at[pl.ds(0, CH)], idxc, semi).wait()

    def issue_g(idxr, idxc, pa, pb, semg):
        pltpu.async_copy(pos_hbm.at[idxr], pa, semg)
        pltpu.async_copy(pos_hbm.at[idxc], pb, semg)

    def wait_g(idxr, idxc, pa, pb, semg):
        pltpu.make_async_copy(pos_hbm.at[idxr], pa, semg).wait()
        pltpu.make_async_copy(pos_hbm.at[idxc], pb, semg).wait()

    def process(t, pa, pb, ob, semw):
        @pl.when(t >= 2)
        def _():
            pltpu.make_async_copy(ob, out_hbm.at[pl.ds(0, CH), :], semw).wait()

        def ebody(e4, carry):
            for i in range(4):
                e = e4 * 4 + i
                ob[e, :] = pa[e, :] - pb[e, :]
            return carry

        lax.fori_loop(0, CH // 4, ebody, 0)
        pltpu.async_copy(ob, out_hbm.at[pl.ds(off(t), CH), :], semw)

    issue_idx(0, idxr0, idxc0, semi0)
    wait_idx(idxr0, idxc0, semi0)
    issue_g(idxr0, idxc0, pa0, pb0, semg0)
    issue_idx(1, idxr1, idxc1, semi1)

    def pair(u, carry):
        t0 = u * 2
        wait_idx(idxr1, idxc1, semi1)
        issue_g(idxr1, idxc1, pa1, pb1, semg1)
        wait_g(idxr0, idxc0, pa0, pb0, semg0)
        process(t0, pa0, pb0, ob0, semw0)

        @pl.when(t0 + 2 < nch)
        def _():
            issue_idx(t0 + 2, idxr0, idxc0, semi0)

        wait_g(idxr1, idxc1, pa1, pb1, semg1)
        process(t0 + 1, pa1, pb1, ob1, semw1)

        @pl.when(t0 + 2 < nch)
        def _():
            wait_idx(idxr0, idxc0, semi0)
            issue_g(idxr0, idxc0, pa0, pb0, semg0)

        @pl.when(t0 + 3 < nch)
        def _():
            issue_idx(t0 + 3, idxr1, idxc1, semi1)

        return carry

    lax.fori_loop(0, npairs, pair, 0)

    @pl.when(nch % 2 == 1)
    def _():
        wait_g(idxr0, idxc0, pa0, pb0, semg0)
        process(npairs * 2, pa0, pb0, ob0, semw0)

    pltpu.make_async_copy(ob0, out_hbm.at[pl.ds(0, CH), :], semw0).wait()
    pltpu.make_async_copy(ob1, out_hbm.at[pl.ds(0, CH), :], semw1).wait()


def _sc0(pos16, row, col):
    f = functools.partial(
        pl.kernel, mesh=_mk_mesh(),
        compiler_params=pltpu.CompilerParams(use_tc_tiling_on_sc=False, needs_layout_passes=False),
        out_type=jax.ShapeDtypeStruct((E, 16), jnp.float32),
        scratch_types=[
            pltpu.VMEM((CH,), jnp.int32), pltpu.VMEM((CH,), jnp.int32),
            pltpu.VMEM((CH,), jnp.int32), pltpu.VMEM((CH,), jnp.int32),
            pltpu.VMEM((CH, 16), jnp.float32), pltpu.VMEM((CH, 16), jnp.float32),
            pltpu.VMEM((CH, 16), jnp.float32), pltpu.VMEM((CH, 16), jnp.float32),
            pltpu.VMEM((CH, 16), jnp.float32), pltpu.VMEM((CH, 16), jnp.float32),
        ] + [pltpu.SemaphoreType.DMA] * 6)(_sc0_body)
    return f(pos16, row, col)


CH1 = 32
NCHUNK1 = E // CH1


def _sc1_body(q_hbm, k_hbm, v_hbm, row_hbm, col_hbm, lb_hbm, zb_hbm, zs_hbm,
              acc_out, s_out,
              idxr0, idxc0, idxr1, idxc1,
              qg0, kg0, vg0, qg1, kg1, vg1,
              lb0, lb1, wb0, wb1, mb0, mb1,
              semi0, semi1, semg0, semg1,
              acc_sp, s_sp):
    cid = lax.axis_index("c")
    sid = lax.axis_index("s")
    w = sid * NC + cid
    base = sid * RPS
    nch = jnp.where(w < (NCHUNK1 % NW), NCHUNK1 // NW + 1, NCHUNK1 // NW)
    npairs = nch // 2

    pltpu.sync_copy(zb_hbm, mb0)
    pltpu.sync_copy(zs_hbm, wb0)
    for (o, s) in _staged_rows(RPS, CH1):
        pltpu.sync_copy(mb0.at[pl.ds(0, s), :], acc_sp.at[pl.ds(base + o, s), :])
        pltpu.sync_copy(wb0.at[pl.ds(0, s), :], s_sp.at[pl.ds(base + o, s), :])
    plsc.subcore_barrier()

    lane = lax.broadcasted_iota(jnp.int32, (16,), 0)

    def off(t):
        return (t * NW + w) * CH1

    def issue_idx(t, idxr, idxc, semi):
        pltpu.async_copy(row_hbm.at[pl.ds(off(t), CH1)], idxr, semi)
        pltpu.async_copy(col_hbm.at[pl.ds(off(t), CH1)], idxc, semi)

    def wait_idx(idxr, idxc, semi):
        pltpu.make_async_copy(row_hbm.at[pl.ds(0, CH1)], idxr, semi).wait()
        pltpu.make_async_copy(col_hbm.at[pl.ds(0, CH1)], idxc, semi).wait()

    def issue_g(t, idxr, idxc, qg, kg, vg, lb, semg):
        pltpu.async_copy(q_hbm.at[idxr], qg, semg)
        pltpu.async_copy(k_hbm.at[idxc], kg, semg)
        pltpu.async_copy(v_hbm.at[idxc], vg, semg)
        pltpu.async_copy(lb_hbm.at[pl.ds(off(t), CH1), :], lb, semg)

    def wait_g(idxr, idxc, qg, kg, vg, lb, semg):
        pltpu.make_async_copy(q_hbm.at[idxr], qg, semg).wait()
        pltpu.make_async_copy(k_hbm.at[idxc], kg, semg).wait()
        pltpu.make_async_copy(v_hbm.at[idxc], vg, semg).wait()
        pltpu.make_async_copy(lb_hbm.at[pl.ds(0, CH1), :], lb, semg).wait()

    def process(qg, kg, vg, lb, wb, mb, idxr):
        def gbody(g, carry):
            erow = g * 16 + lane
            for h in range(NH):
                acc = jnp.zeros((16,), jnp.float32)
                for d in range(DH):
                    cidx = jnp.full((16,), h * DH + d, jnp.int32)
                    qv = plsc.load_gather(qg, [erow, cidx])
                    kv = plsc.load_gather(kg, [erow, cidx])
                    acc = acc + qv * kv
                hcol = jnp.full((16,), h, jnp.int32)
                lbv = plsc.load_gather(lb, [erow, hcol])
                wh = jnp.exp(acc * INV_SQRT_DH + lbv)
                plsc.store_scatter(wb, [erow, hcol], wh)
                for d in range(DH):
                    cidx = jnp.full((16,), h * DH + d, jnp.int32)
                    vv = plsc.load_gather(vg, [erow, cidx])
                    plsc.store_scatter(mb, [erow, cidx], wh * vv)
            return carry

        lax.fori_loop(0, CH1 // 16, gbody, 0)
        pltpu.sync_copy(mb, acc_sp.at[idxr], add=True)
        pltpu.sync_copy(wb, s_sp.at[idxr], add=True)

    issue_idx(0, idxr0, idxc0, semi0)
    wait_idx(idxr0, idxc0, semi0)
    issue_g(0, idxr0, idxc0, qg0, kg0, vg0, lb0, semg0)
    issue_idx(1, idxr1, idxc1, semi1)

    def pair(u, carry):
        t0 = u * 2
        wait_idx(idxr1, idxc1, semi1)
        issue_g(t0 + 1, idxr1, idxc1, qg1, kg1, vg1, lb1, semg1)
        wait_g(idxr0, idxc0, qg0, kg0, vg0, lb0, semg0)
        process(qg0, kg0, vg0, lb0, wb0, mb0, idxr0)

        @pl.when(t0 + 2 < nch)
        def _():
            issue_idx(t0 + 2, idxr0, idxc0, semi0)

        wait_g(idxr1, idxc1, qg1, kg1, vg1, lb1, semg1)
        process(qg1, kg1, vg1, lb1, wb1, mb1, idxr1)

        @pl.when(t0 + 2 < nch)
        def _():
            wait_idx(idxr0, idxc0, semi0)
            issue_g(t0 + 2, idxr0, idxc0, qg0, kg0, vg0, lb0, semg0)

        @pl.when(t0 + 3 < nch)
        def _():
            issue_idx(t0 + 3, idxr1, idxc1, semi1)

        return carry

    lax.fori_loop(0, npairs, pair, 0)

    @pl.when(nch % 2 == 1)
    def _():
        wait_g(idxr0, idxc0, qg0, kg0, vg0, lb0, semg0)
        process(qg0, kg0, vg0, lb0, wb0, mb0, idxr0)

    plsc.subcore_barrier()

    for (o, s) in _staged_rows(RPS, CH1):
        pltpu.sync_copy(acc_sp.at[pl.ds(base + o, s), :], mb0.at[pl.ds(0, s), :])
        pltpu.sync_copy(mb0.at[pl.ds(0, s), :], acc_out.at[cid, pl.ds(base + o, s), :])
        pltpu.sync_copy(s_sp.at[pl.ds(base + o, s), :], wb0.at[pl.ds(0, s), :])
        pltpu.sync_copy(wb0.at[pl.ds(0, s), :], s_out.at[cid, pl.ds(base + o, s), :])


def _sc1(q, k, v, row, col, lb16, zb, zs):
    f = functools.partial(
        pl.kernel, mesh=_mk_mesh(),
        compiler_params=pltpu.CompilerParams(use_tc_tiling_on_sc=False, needs_layout_passes=False),
        out_type=(jax.ShapeDtypeStruct((NC, N, D), jnp.float32),
                  jax.ShapeDtypeStruct((NC, N, 16), jnp.float32)),
        scratch_types=[
            pltpu.VMEM((CH1,), jnp.int32), pltpu.VMEM((CH1,), jnp.int32),
            pltpu.VMEM((CH1,), jnp.int32), pltpu.VMEM((CH1,), jnp.int32),
            pltpu.VMEM((CH1, D), jnp.float32), pltpu.VMEM((CH1, D), jnp.float32),
            pltpu.VMEM((CH1, D), jnp.float32), pltpu.VMEM((CH1, D), jnp.float32),
            pltpu.VMEM((CH1, D), jnp.float32), pltpu.VMEM((CH1, D), jnp.float32),
            pltpu.VMEM((CH1, 16), jnp.float32), pltpu.VMEM((CH1, 16), jnp.float32),
            pltpu.VMEM((CH1, 16), jnp.float32), pltpu.VMEM((CH1, 16), jnp.float32),
            pltpu.VMEM((CH1, D), jnp.float32), pltpu.VMEM((CH1, D), jnp.float32),
        ] + [pltpu.SemaphoreType.DMA] * 4 + [
            pltpu.VMEM_SHARED((N, D), jnp.float32),
            pltpu.VMEM_SHARED((N, 16), jnp.float32),
        ])(_sc1_body)
    return f(q, k, v, row, col, lb16, zb, zs)


def _sc2_body(b_hbm, c1_hbm, c2_hbm, row_hbm, col_hbm, bs_out, gs_out,
              idxr0, idxc0, idxr1, idxc1,
              b10, b20, g10, g20, b11, b21, g11, g21,
              bo0, go0, bo1, go1,
              semi0, semi1, semg0, semg1, semw0, semw1):
    w = _wid()
    nch = jnp.where(w < (NCHUNK % NW), NCHUNK // NW + 1, NCHUNK // NW)
    npairs = nch // 2

    def off(t):
        return (t * NW + w) * CH

    def issue_idx(t, idxr, idxc, semi):
        pltpu.async_copy(row_hbm.at[pl.ds(off(t), CH)], idxr, semi)
        pltpu.async_copy(col_hbm.at[pl.ds(off(t), CH)], idxc, semi)

    def wait_idx(idxr, idxc, semi):
        pltpu.make_async_copy(row_hbm.at[pl.ds(0, CH)], idxr, semi).wait()
        pltpu.make_async_copy(col_hbm.at[pl.ds(0, CH)], idxc, semi).wait()

    def issue_g(idxr, idxc, b1, b2, g1, g2, semg):
        pltpu.async_copy(b_hbm.at[idxr], b1, semg)
        pltpu.async_copy(b_hbm.at[idxc], b2, semg)
        pltpu.async_copy(c1_hbm.at[idxr], g1, semg)
        pltpu.async_copy(c2_hbm.at[idxc], g2, semg)

    def wait_g(idxr, idxc, b1, b2, g1, g2, semg):
        pltpu.make_async_copy(b_hbm.at[idxr], b1, semg).wait()
        pltpu.make_async_copy(b_hbm.at[idxc], b2, semg).wait()
        pltpu.make_async_copy(c1_hbm.at[idxr], g1, semg).wait()
        pltpu.make_async_copy(c2_hbm.at[idxc], g2, semg).wait()

    def process(t, b1, b2, g1, g2, bo, go, semw):
        @pl.when(t >= 2)
        def _():
            pltpu.make_async_copy(bo, bs_out.at[pl.ds(0, CH), :], semw).wait()
            pltpu.make_async_copy(go, gs_out.at[pl.ds(0, CH), :], semw).wait()

        def ebody(e4, carry):
            for i in range(4):
                e = e4 * 4 + i
                bo[e, :] = b1[e, :] + b2[e, :]
                for u in range(D // 16):
                    go[e, pl.ds(u * 16, 16)] = (g1[e, pl.ds(u * 16, 16)]
                                                + g2[e, pl.ds(u * 16, 16)])
            return carry

        lax.fori_loop(0, CH // 4, ebody, 0)
        pltpu.async_copy(bo, bs_out.at[pl.ds(off(t), CH), :], semw)
        pltpu.async_copy(go, gs_out.at[pl.ds(off(t), CH), :], semw)

    issue_idx(0, idxr0, idxc0, semi0)
    wait_idx(idxr0, idxc0, semi0)
    issue_g(idxr0, idxc0, b10, b20, g10, g20, semg0)
    issue_idx(1, idxr1, idxc1, semi1)

    def pair(u, carry):
        t0 = u * 2
        wait_idx(idxr1, idxc1, semi1)
        issue_g(idxr1, idxc1, b11, b21, g11, g21, semg1)
        wait_g(idxr0, idxc0, b10, b20, g10, g20, semg0)
        process(t0, b10, b20, g10, g20, bo0, go0, semw0)

        @pl.when(t0 + 2 < nch)
        def _():
            issue_idx(t0 + 2, idxr0, idxc0, semi0)

        wait_g(idxr1, idxc1, b11, b21, g11, g21, semg1)
        process(t0 + 1, b11, b21, g11, g21, bo1, go1, semw1)

        @pl.when(t0 + 2 < nch)
        def _():
            wait_idx(idxr0, idxc0, semi0)
            issue_g(idxr0, idxc0, b10, b20, g10, g20, semg0)

        @pl.when(t0 + 3 < nch)
        def _():
            issue_idx(t0 + 3, idxr1, idxc1, semi1)

        return carry

    lax.fori_loop(0, npairs, pair, 0)

    @pl.when(nch % 2 == 1)
    def _():
        wait_g(idxr0, idxc0, b10, b20, g10, g20, semg0)
        process(npairs * 2, b10, b20, g10, g20, bo0, go0, semw0)

    pltpu.make_async_copy(bo0, bs_out.at[pl.ds(0, CH), :], semw0).wait()
    pltpu.make_async_copy(go0, gs_out.at[pl.ds(0, CH), :], semw0).wait()
    pltpu.make_async_copy(bo1, bs_out.at[pl.ds(0, CH), :], semw1).wait()
    pltpu.make_async_copy(go1, gs_out.at[pl.ds(0, CH), :], semw1).wait()


def _sc2(b_tab, c1, c2, row, col):
    f = functools.partial(
        pl.kernel, mesh=_mk_mesh(),
        compiler_params=pltpu.CompilerParams(use_tc_tiling_on_sc=False, needs_layout_passes=False),
        out_type=(jax.ShapeDtypeStruct((E, 16), jnp.float32),
                  jax.ShapeDtypeStruct((E, D), jnp.float32)),
        scratch_types=[
            pltpu.VMEM((CH,), jnp.int32), pltpu.VMEM((CH,), jnp.int32),
            pltpu.VMEM((CH,), jnp.int32), pltpu.VMEM((CH,), jnp.int32),
            pltpu.VMEM((CH, 16), jnp.float32), pltpu.VMEM((CH, 16), jnp.float32),
            pltpu.VMEM((CH, D), jnp.float32), pltpu.VMEM((CH, D), jnp.float32),
            pltpu.VMEM((CH, 16), jnp.float32), pltpu.VMEM((CH, 16), jnp.float32),
            pltpu.VMEM((CH, D), jnp.float32), pltpu.VMEM((CH, D), jnp.float32),
            pltpu.VMEM((CH, 16), jnp.float32), pltpu.VMEM((CH, D), jnp.float32),
            pltpu.VMEM((CH, 16), jnp.float32), pltpu.VMEM((CH, D), jnp.float32),
        ] + [pltpu.SemaphoreType.DMA] * 6)(_sc2_body)
    return f(b_tab, c1, c2, row, col)


def _sc3_body(con_hbm, row_hbm, zs_hbm, agg_out,
              idxr0, idxr1, cb0, cb1, sb, semi0, semi1, agg_sp):
    cid = lax.axis_index("c")
    sid = lax.axis_index("s")
    w = sid * NC + cid
    base = sid * RPS
    nch = jnp.where(w < (NCHUNK % NW), NCHUNK // NW + 1, NCHUNK // NW)
    npairs = nch // 2

    pltpu.sync_copy(zs_hbm, sb)
    pltpu.sync_copy(sb, agg_sp.at[pl.ds(base, RPS), :])
    plsc.subcore_barrier()

    def off(t):
        return (t * NW + w) * CH

    def issue(t, idxr, cb, semi):
        pltpu.async_copy(row_hbm.at[pl.ds(off(t), CH)], idxr, semi)
        pltpu.async_copy(con_hbm.at[pl.ds(off(t), CH), :], cb, semi)

    def wait(idxr, cb, semi):
        pltpu.make_async_copy(row_hbm.at[pl.ds(0, CH)], idxr, semi).wait()
        pltpu.make_async_copy(con_hbm.at[pl.ds(0, CH), :], cb, semi).wait()

    issue(0, idxr0, cb0, semi0)
    issue(1, idxr1, cb1, semi1)

    def pair(u, carry):
        t0 = u * 2
        wait(idxr0, cb0, semi0)
        pltpu.sync_copy(cb0, agg_sp.at[idxr0], add=True)

        @pl.when(t0 + 2 < nch)
        def _():
            issue(t0 + 2, idxr0, cb0, semi0)

        wait(idxr1, cb1, semi1)
        pltpu.sync_copy(cb1, agg_sp.at[idxr1], add=True)

        @pl.when(t0 + 3 < nch)
        def _():
            issue(t0 + 3, idxr1, cb1, semi1)

        return carry

    lax.fori_loop(0, npairs, pair, 0)

    @pl.when(nch % 2 == 1)
    def _():
        wait(idxr0, cb0, semi0)
        pltpu.sync_copy(cb0, agg_sp.at[idxr0], add=True)

    plsc.subcore_barrier()

    pltpu.sync_copy(agg_sp.at[pl.ds(base, RPS), :], sb)
    pltpu.sync_copy(sb, agg_out.at[cid, pl.ds(base, RPS), :])


def _sc3(contrib, row, zs):
    f = functools.partial(
        pl.kernel, mesh=_mk_mesh(),
        compiler_params=pltpu.CompilerParams(use_tc_tiling_on_sc=False, needs_layout_passes=False),
        out_type=jax.ShapeDtypeStruct((NC, N, 16), jnp.float32),
        scratch_types=[
            pltpu.VMEM((CH,), jnp.int32), pltpu.VMEM((CH,), jnp.int32),
            pltpu.VMEM((CH, 16), jnp.float32), pltpu.VMEM((CH, 16), jnp.float32),
            pltpu.VMEM((RPS, 16), jnp.float32),
        ] + [pltpu.SemaphoreType.DMA] * 2 + [
            pltpu.VMEM_SHARED((N, 16), jnp.float32),
        ])(_sc3_body)
    return f(contrib, row, zs)


def _ln(x, eps=1e-6):
    m = jnp.mean(x, axis=-1, keepdims=True)
    v = jnp.mean((x - m) * (x - m), axis=-1, keepdims=True)
    return (x - m) / jnp.sqrt(v + eps)


def _silu(x):
    return x / (1.0 + jnp.exp(-x))


def _mod(x, sh, sc):
    return x * (1 + sc) + sh


def _dot(a, b):
    return jax.lax.dot_general(a, b, (((1,), (0,)), ((), ())),
                               preferred_element_type=jnp.float32)


def _bcast_spec(arr):
    nd = arr.ndim
    return pl.BlockSpec(arr.shape, lambda i: (0,) * nd)


def _rows_spec(blk, cols):
    return pl.BlockSpec((blk, cols), lambda i: (i, 0))


_TC_PARAMS = pltpu.CompilerParams(dimension_semantics=("arbitrary",))


def _tca_body(h, nte, wtm, btm, wq, wk, wv, q_o, k_o, v_o,
              g1_o, g2_o, g3_o, g4_o):
    tm = _dot(_silu(nte[...]), wtm[...]) + btm[...]
    hm = _mod(_ln(h[...]), tm[:, 0:D], tm[:, D:2 * D])
    q_o[...] = _dot(hm, wq[...])
    k_o[...] = _dot(hm, wk[...])
    v_o[...] = _dot(hm, wv[...])
    g1_o[...] = tm[:, 2 * D:3 * D]
    g2_o[...] = tm[:, 3 * D:4 * D]
    g3_o[...] = tm[:, 4 * D:5 * D]
    g4_o[...] = tm[:, 5 * D:6 * D]


def _tca(h, nte, p):
    blk = 1000
    wtm, btm = p['node_tm_W'], p['node_tm_b'].reshape(1, -1)
    outs = [jax.ShapeDtypeStruct((N, D), jnp.float32)] * 7
    return pl.pallas_call(
        _tca_body,
        grid=(N // blk,),
        in_specs=[_rows_spec(blk, D), _rows_spec(blk, D),
                  _bcast_spec(wtm), _bcast_spec(btm),
                  _bcast_spec(p['Wq']), _bcast_spec(p['Wk']), _bcast_spec(p['Wv'])],
        out_specs=[_rows_spec(blk, D)] * 7,
        out_shape=outs,
        compiler_params=_TC_PARAMS,
    )(h, nte, wtm, btm, p['Wq'], p['Wk'], p['Wv'])


def _tcb_body(pdiff, ea, ete, ex8, wd, we2, be2, wtm, btm, wl, wx8,
              misc_o, g1_o, g2_o, g3_o, g4_o):
    pd = pdiff[...]
    dist = jnp.sum(pd * pd, axis=-1, keepdims=True)
    e2 = dist * wd[...] + _dot(ea[...], we2[...]) + be2[...]
    tm = _dot(_silu(ete[...]), wtm[...]) + btm[...]
    emod = _mod(_ln(e2), tm[:, 0:16], tm[:, 16:32])
    lb = _dot(emod, wl[...]) + _dot(ex8[...], wx8[...])
    blk = lb.shape[0]
    misc_o[...] = jnp.concatenate(
        [lb, dist, jnp.zeros((blk, 7), jnp.float32)], axis=1)
    g1_o[...] = tm[:, 32:48]
    g2_o[...] = tm[:, 48:64]
    g3_o[...] = tm[:, 64:80]
    g4_o[...] = tm[:, 80:96]


def _tcb(pdiff, ea, ete, ex8, p):
    blk = 2000
    wd = p['edge_emb_W'][0:1]
    we2 = p['edge_emb_W'][1:]
    be2 = p['edge_emb_b'].reshape(1, -1)
    wtm, btm = p['edge_tm_W'], p['edge_tm_b'].reshape(1, -1)
    wx8 = jnp.pad(p['Wx'], ((0, 6), (0, 0)))
    outs = [jax.ShapeDtypeStruct((E, 16), jnp.float32)] * 5
    return pl.pallas_call(
        _tcb_body,
        grid=(E // blk,),
        in_specs=[_rows_spec(blk, 16), _rows_spec(blk, 16),
                  _rows_spec(blk, D), _rows_spec(blk, 8),
                  _bcast_spec(wd), _bcast_spec(we2), _bcast_spec(be2),
                  _bcast_spec(wtm), _bcast_spec(btm),
                  _bcast_spec(p['We']), _bcast_spec(wx8)],
        out_specs=[_rows_spec(blk, 16)] * 5,
        out_shape=outs,
        compiler_params=_TC_PARAMS,
    )(pdiff, ea, ete, ex8, wd, we2, be2, wtm, btm, p['We'], wx8)


def _tcc_body(acc3, s3, h, nmask, g1, g2, g3, g4, sexp, wo, n2e,
              ff1, fb1, ff2, fb2, eqw1, eqw2,
              hout_o, b_o, c1_o, c2_o):
    acc = acc3[0] + acc3[1]
    sv = (s3[0] + s3[1])[:, 0:8]
    se = jax.lax.dot_general(sv, sexp[...], (((1,), (0,)), ((), ())),
                             precision=jax.lax.Precision.HIGHEST,
                             preferred_element_type=jnp.float32)
    att = _dot(acc / (se + 1e-16), wo[...])
    b_o[...] = _dot(att, n2e[...])
    h_node = h[...] + g1[...] * att
    hml = _mod(_ln(h_node), g2[...], g3[...]) * nmask[...]
    ffn = _dot(_silu(_dot(hml, ff1[...]) + fb1[...]), ff2[...]) + fb2[...]
    h_out = (hml + g4[...] * ffn) * nmask[...]
    hout_o[...] = h_out
    c1_o[...] = _dot(h_out, eqw1[...])
    c2_o[...] = _dot(h_out, eqw2[...])


def _tcc(acc_p, s_p, h, nmask, g1, g2, g3, g4, p):
    blk = 1000
    sexp = jnp.kron(jnp.eye(8, dtype=jnp.float32),
                    jnp.ones((1, DH), jnp.float32))
    fb1 = p['ff1_b'].reshape(1, -1)
    fb2 = p['ff2_b'].reshape(1, -1)
    eqw1 = p['eq_in_W'][0:D]
    eqw2 = p['eq_in_W'][D:2 * D]
    outs = [jax.ShapeDtypeStruct((N, D), jnp.float32),
            jax.ShapeDtypeStruct((N, 16), jnp.float32),
            jax.ShapeDtypeStruct((N, D), jnp.float32),
            jax.ShapeDtypeStruct((N, D), jnp.float32)]
    return pl.pallas_call(
        _tcc_body,
        grid=(N // blk,),
        in_specs=[pl.BlockSpec((NC, blk, D), lambda i: (0, i, 0)),
                  pl.BlockSpec((NC, blk, 16), lambda i: (0, i, 0)),
                  _rows_spec(blk, D), _rows_spec(blk, 1),
                  _rows_spec(blk, D), _rows_spec(blk, D),
                  _rows_spec(blk, D), _rows_spec(blk, D),
                  _bcast_spec(sexp), _bcast_spec(p['Wo']), _bcast_spec(p['n2e_W']),
                  _bcast_spec(p['ff1_W']), _bcast_spec(fb1),
                  _bcast_spec(p['ff2_W']), _bcast_spec(fb2),
                  _bcast_spec(eqw1), _bcast_spec(eqw2)],
        out_specs=[_rows_spec(blk, D), _rows_spec(blk, 16),
                   _rows_spec(blk, D), _rows_spec(blk, D)],
        out_shape=outs,
        compiler_params=_TC_PARAMS,
    )(acc_p, s_p, h, nmask, g1, g2, g3, g4, sexp, p['Wo'], p['n2e_W'],
      p['ff1_W'], fb1, p['ff2_W'], fb2, eqw1, eqw2)


def _tcde_body(ea, g1, g2, g3, g4, bsum, gsum, misc, pdiff, ete, ex8,
               n2eb, ff3, fb3, ff4, fb4, wtm, btm, w3, w4, eqb,
               c1w, c1b, c2w, cscale,
               heo_o, con_o):
    he = ea[...] + g1[...] * (bsum[...] + n2eb[...])
    he = _mod(_ln(he), g2[...], g3[...])
    ffe = _dot(_silu(_dot(he, ff3[...]) + fb3[...]), ff4[...]) + fb4[...]
    heo = he + g4[...] * ffe
    heo_o[...] = heo
    tm = _dot(_silu(ete[...]), wtm[...]) + btm[...]
    dist = misc[:, 8:9]
    lin = gsum[...] + _dot(heo, w3[...]) + dist * w4[...] + eqb[...]
    inv = _mod(_ln(lin), tm[:, 0:D], tm[:, D:2 * D])
    u = jnp.tanh(_dot(_silu(_dot(inv, c1w[...]) + c1b[...]), c2w[...]))
    blk = u.shape[0]
    adjs = jnp.concatenate([jnp.ones((blk, 1), jnp.float32), ex8[:, 0:7]],
                           axis=1)
    invm = jnp.sum(u * adjs, axis=-1, keepdims=True) * (1.0 / 3.0)
    nrm = jnp.sqrt(dist)
    cdf = pdiff[...] / jnp.maximum(nrm, 1e-8) * cscale[...]
    con_o[...] = cdf * invm


def _tcde(ea, g1, g2, g3, g4, bsum, gsum, misc, pdiff, ete, ex8, p):
    blk = 2000
    n2eb = p['n2e_b'].reshape(1, -1)
    fb3 = p['ff3_b'].reshape(1, -1)
    fb4 = p['ff4_b'].reshape(1, -1)
    wtm, btm = p['eq_tm_W'], p['eq_tm_b'].reshape(1, -1)
    w3 = p['eq_in_W'][2 * D:2 * D + 16]
    w4 = p['eq_in_W'][2 * D + 16:2 * D + 17]
    eqb = p['eq_in_b'].reshape(1, -1)
    c1b = p['eq_c1_b'].reshape(1, -1)
    c2w = jnp.pad(p['eq_c2_W'], ((0, 0), (0, 5)))
    cscale = p['coors_scale'].reshape(1, 1)
    outs = [jax.ShapeDtypeStruct((E, 16), jnp.float32),
            jax.ShapeDtypeStruct((E, 16), jnp.float32)]
    return pl.pallas_call(
        _tcde_body,
        grid=(E // blk,),
        in_specs=[_rows_spec(blk, 16), _rows_spec(blk, 16), _rows_spec(blk, 16),
                  _rows_spec(blk, 16), _rows_spec(blk, 16), _rows_spec(blk, 16),
                  _rows_spec(blk, D), _rows_spec(blk, 16), _rows_spec(blk, 16),
                  _rows_spec(blk, D), _rows_spec(blk, 8),
                  _bcast_spec(n2eb), _bcast_spec(p['ff3_W']), _bcast_spec(fb3),
                  _bcast_spec(p['ff4_W']), _bcast_spec(fb4),
                  _bcast_spec(wtm), _bcast_spec(btm), _bcast_spec(w3),
                  _bcast_spec(w4), _bcast_spec(eqb),
                  _bcast_spec(p['eq_c1_W']), _bcast_spec(c1b), _bcast_spec(c2w),
                  _bcast_spec(cscale)],
        out_specs=[_rows_spec(blk, 16), _rows_spec(blk, 16)],
        out_shape=outs,
        compiler_params=_TC_PARAMS,
    )(ea, g1, g2, g3, g4, bsum, gsum, misc, pdiff, ete, ex8,
      n2eb, p['ff3_W'], fb3, p['ff4_W'], fb4, wtm, btm, w3, w4, eqb,
      p['eq_c1_W'], c1b, c2w, cscale)


def _tcf_body(pos16, agg3, out_o):
    out_o[...] = pos16[...] + agg3[0] + agg3[1]


def _tcf(pos16, agg_p):
    blk = 1000
    return pl.pallas_call(
        _tcf_body,
        grid=(N // blk,),
        in_specs=[_rows_spec(blk, 16),
                  pl.BlockSpec((NC, blk, 16), lambda i: (0, i, 0))],
        out_specs=_rows_spec(blk, 16),
        out_shape=jax.ShapeDtypeStruct((N, 16), jnp.float32),
        compiler_params=_TC_PARAMS,
    )(pos16, agg_p)


def kernel(pos, h, edge_attr, edge_index, node_mask, extra_heads,
           node_time_emb, edge_time_emb, params):
    p = params
    row = edge_index[0]
    col = edge_index[1]
    pos16 = jnp.pad(pos, ((0, 0), (0, 13)))
    ex8 = jnp.pad(extra_heads, ((0, 0), (0, 6)))
    zb = jnp.zeros((CH1, D), jnp.float32)
    zs1 = jnp.zeros((CH1, 16), jnp.float32)
    zs3 = jnp.zeros((RPS, 16), jnp.float32)

    pdiff = _sc0(pos16, row, col)
    q, k, v, g1, g2, g3, g4 = _tca(h, node_time_emb, p)
    misc, e1, e2m, e3, e4 = _tcb(pdiff, edge_attr, edge_time_emb, ex8, p)
    acc_p, s_p = _sc1(q, k, v, row, col, misc, zb, zs1)
    h_out, b_tab, c1, c2 = _tcc(acc_p, s_p, h, node_mask, g1, g2, g3, g4, p)
    bsum, gsum = _sc2(b_tab, c1, c2, row, col)
    h_edge_out, contrib = _tcde(edge_attr, e1, e2m, e3, e4, bsum, gsum,
                                misc, pdiff, edge_time_emb, ex8, p)
    agg_p = _sc3(contrib, row, zs3)
    pos_out16 = _tcf(pos16, agg_p)
    return h_out, h_edge_out, pos_out16[:, 0:3]


# speedup vs baseline: 1.7038x; 1.7038x over previous
import functools

import jax
import jax.numpy as jnp
import numpy as np
from jax import lax
from jax.experimental import pallas as pl
from jax.experimental.pallas import tpu as pltpu
from jax.experimental.pallas import tpu_sc as plsc

N = 10000
E = 160000
D = 128
ED = 16
NH = 8
DH = 16
NC = 2
NS = 16
NW = NC * NS
CH = 128
NCHUNK = E // CH
RPS = N // NS
INV_SQRT_DH = 1.0 / np.sqrt(DH)

def _mk_mesh():
    return plsc.VectorSubcoreMesh(core_axis_name="c", subcore_axis_name="s",
                                  num_cores=NC, num_subcores=NS)


def _wid():
    return lax.axis_index("s") * NC + lax.axis_index("c")


def _chunk_loop(body, nchunk=NCHUNK):
    w = _wid()
    nch = jnp.where(w < (nchunk % NW), nchunk // NW + 1, nchunk // NW)

    def tbody(t, carry):
        body(t * NW + w)
        return carry

    lax.fori_loop(0, nch, tbody, 0)


def _staged_rows(n, step=64):
    out = []
    o = 0
    while o < n:
        s = min(step, n - o)
        out.append((o, s))
        o += s
    return out


def _sc0_body(pos_hbm, row_hbm, col_hbm, out_hbm,
              idxr0, idxc0, idxr1, idxc1, pa0, pb0, pa1, pb1, ob0, ob1,
              semi0, semi1, semg0, semg1, semw0, semw1):
    w = _wid()
    nch = jnp.where(w < (NCHUNK % NW), NCHUNK // NW + 1, NCHUNK // NW)
    npairs = nch // 2

    def off(t):
        return (t * NW + w) * CH

    def issue_idx(t, idxr, idxc, semi):
        pltpu.async_copy(row_hbm.at[pl.ds(off(t), CH)], idxr, semi)
        pltpu.async_copy(col_hbm.at[pl.ds(off(t), CH)], idxc, semi)

    def wait_idx(idxr, idxc, semi):
        pltpu.make_async_copy(row_hbm.at[pl.ds(0, CH)], idxr, semi).wait()
        pltpu.make_async_copy(col_hbm.at[pl.ds(0, CH)], idxc, semi).wait()

    def issue_g(idxr, idxc, pa, pb, semg):
        pltpu.async_copy(pos_hbm.at[idxr], pa, semg)
        pltpu.async_copy(pos_hbm.at[idxc], pb, semg)

    def wait_g(idxr, idxc, pa, pb, semg):
        pltpu.make_async_copy(pos_hbm.at[idxr], pa, semg).wait()
        pltpu.make_async_copy(pos_hbm.at[idxc], pb, semg).wait()

    def process(t, pa, pb, ob, semw):
        @pl.when(t >= 2)
        def _():
            pltpu.make_async_copy(ob, out_hbm.at[pl.ds(0, CH), :], semw).wait()

        def ebody(e4, carry):
            for i in range(4):
                e = e4 * 4 + i
                ob[e, :] = pa[e, :] - pb[e, :]
            return carry

        lax.fori_loop(0, CH // 4, ebody, 0)
        pltpu.async_copy(ob, out_hbm.at[pl.ds(off(t), CH), :], semw)

    issue_idx(0, idxr0, idxc0, semi0)
    wait_idx(idxr0, idxc0, semi0)
    issue_g(idxr0, idxc0, pa0, pb0, semg0)
    issue_idx(1, idxr1, idxc1, semi1)

    def pair(u, carry):
        t0 = u * 2
        wait_idx(idxr1, idxc1, semi1)
        issue_g(idxr1, idxc1, pa1, pb1, semg1)
        wait_g(idxr0, idxc0, pa0, pb0, semg0)
        process(t0, pa0, pb0, ob0, semw0)

        @pl.when(t0 + 2 < nch)
        def _():
            issue_idx(t0 + 2, idxr0, idxc0, semi0)

        wait_g(idxr1, idxc1, pa1, pb1, semg1)
        process(t0 + 1, pa1, pb1, ob1, semw1)

        @pl.when(t0 + 2 < nch)
        def _():
            wait_idx(idxr0, idxc0, semi0)
            issue_g(idxr0, idxc0, pa0, pb0, semg0)

        @pl.when(t0 + 3 < nch)
        def _():
            issue_idx(t0 + 3, idxr1, idxc1, semi1)

        return carry

    lax.fori_loop(0, npairs, pair, 0)

    @pl.when(nch % 2 == 1)
    def _():
        wait_g(idxr0, idxc0, pa0, pb0, semg0)
        process(npairs * 2, pa0, pb0, ob0, semw0)

    pltpu.make_async_copy(ob0, out_hbm.at[pl.ds(0, CH), :], semw0).wait()
    pltpu.make_async_copy(ob1, out_hbm.at[pl.ds(0, CH), :], semw1).wait()


def _sc0(pos16, row, col):
    f = functools.partial(
        pl.kernel, mesh=_mk_mesh(),
        compiler_params=pltpu.CompilerParams(use_tc_tiling_on_sc=False, needs_layout_passes=False),
        out_type=jax.ShapeDtypeStruct((E, 16), jnp.float32),
        scratch_types=[
            pltpu.VMEM((CH,), jnp.int32), pltpu.VMEM((CH,), jnp.int32),
            pltpu.VMEM((CH,), jnp.int32), pltpu.VMEM((CH,), jnp.int32),
            pltpu.VMEM((CH, 16), jnp.float32), pltpu.VMEM((CH, 16), jnp.float32),
            pltpu.VMEM((CH, 16), jnp.float32), pltpu.VMEM((CH, 16), jnp.float32),
            pltpu.VMEM((CH, 16), jnp.float32), pltpu.VMEM((CH, 16), jnp.float32),
        ] + [pltpu.SemaphoreType.DMA] * 6)(_sc0_body)
    return f(pos16, row, col)


CH1 = 32
NCHUNK1 = E // CH1


def _sc1_body(q_hbm, k_hbm, v_hbm, row_hbm, col_hbm, lb_hbm, zb_hbm, zs_hbm,
              acc_out, s_out,
              idxr0, idxc0, idxr1, idxc1,
              qg0, kg0, vg0, qg1, kg1, vg1,
              lb0, lb1, wb0, wb1, mb0, mb1,
              semi0, semi1, semg0, semg1,
              acc_sp, s_sp):
    cid = lax.axis_index("c")
    sid = lax.axis_index("s")
    w = sid * NC + cid
    base = sid * RPS
    nch = jnp.where(w < (NCHUNK1 % NW), NCHUNK1 // NW + 1, NCHUNK1 // NW)
    npairs = nch // 2

    pltpu.sync_copy(zb_hbm, mb0)
    pltpu.sync_copy(zs_hbm, wb0)
    for (o, s) in _staged_rows(RPS, CH1):
        pltpu.sync_copy(mb0.at[pl.ds(0, s), :], acc_sp.at[pl.ds(base + o, s), :])
        pltpu.sync_copy(wb0.at[pl.ds(0, s), :], s_sp.at[pl.ds(base + o, s), :])
    plsc.subcore_barrier()

    lane = lax.broadcasted_iota(jnp.int32, (16,), 0)

    def off(t):
        return (t * NW + w) * CH1

    def issue_idx(t, idxr, idxc, semi):
        pltpu.async_copy(row_hbm.at[pl.ds(off(t), CH1)], idxr, semi)
        pltpu.async_copy(col_hbm.at[pl.ds(off(t), CH1)], idxc, semi)

    def wait_idx(idxr, idxc, semi):
        pltpu.make_async_copy(row_hbm.at[pl.ds(0, CH1)], idxr, semi).wait()
        pltpu.make_async_copy(col_hbm.at[pl.ds(0, CH1)], idxc, semi).wait()

    def issue_g(t, idxr, idxc, qg, kg, vg, lb, semg):
        pltpu.async_copy(q_hbm.at[idxr], qg, semg)
        pltpu.async_copy(k_hbm.at[idxc], kg, semg)
        pltpu.async_copy(v_hbm.at[idxc], vg, semg)
        pltpu.async_copy(lb_hbm.at[pl.ds(off(t), CH1), :], lb, semg)

    def wait_g(idxr, idxc, qg, kg, vg, lb, semg):
        pltpu.make_async_copy(q_hbm.at[idxr], qg, semg).wait()
        pltpu.make_async_copy(k_hbm.at[idxc], kg, semg).wait()
        pltpu.make_async_copy(v_hbm.at[idxc], vg, semg).wait()
        pltpu.make_async_copy(lb_hbm.at[pl.ds(0, CH1), :], lb, semg).wait()

    def process(qg, kg, vg, lb, wb, mb, idxr):
        def ebody(e, carry):
            lvec = jnp.zeros((16,), jnp.float32)
            for h in range(NH):
                dh = jnp.sum(qg[e, pl.ds(h * DH, DH)] * kg[e, pl.ds(h * DH, DH)])
                lvec = jnp.where(lane == h, dh, lvec)
            wv = jnp.exp(lvec * INV_SQRT_DH + lb[e, :])
            wb[e, :] = wv
            for h in range(NH):
                mb[e, pl.ds(h * DH, DH)] = wv[h] * vg[e, pl.ds(h * DH, DH)]
            return carry

        lax.fori_loop(0, CH1, ebody, 0)
        pltpu.sync_copy(mb, acc_sp.at[idxr], add=True)
        pltpu.sync_copy(wb, s_sp.at[idxr], add=True)

    issue_idx(0, idxr0, idxc0, semi0)
    wait_idx(idxr0, idxc0, semi0)
    issue_g(0, idxr0, idxc0, qg0, kg0, vg0, lb0, semg0)
    issue_idx(1, idxr1, idxc1, semi1)

    def pair(u, carry):
        t0 = u * 2
        wait_idx(idxr1, idxc1, semi1)
        issue_g(t0 + 1, idxr1, idxc1, qg1, kg1, vg1, lb1, semg1)
        wait_g(idxr0, idxc0, qg0, kg0, vg0, lb0, semg0)
        process(qg0, kg0, vg0, lb0, wb0, mb0, idxr0)

        @pl.when(t0 + 2 < nch)
        def _():
            issue_idx(t0 + 2, idxr0, idxc0, semi0)

        wait_g(idxr1, idxc1, qg1, kg1, vg1, lb1, semg1)
        process(qg1, kg1, vg1, lb1, wb1, mb1, idxr1)

        @pl.when(t0 + 2 < nch)
        def _():
            wait_idx(idxr0, idxc0, semi0)
            issue_g(t0 + 2, idxr0, idxc0, qg0, kg0, vg0, lb0, semg0)

        @pl.when(t0 + 3 < nch)
        def _():
            issue_idx(t0 + 3, idxr1, idxc1, semi1)

        return carry

    lax.fori_loop(0, npairs, pair, 0)

    @pl.when(nch % 2 == 1)
    def _():
        wait_g(idxr0, idxc0, qg0, kg0, vg0, lb0, semg0)
        process(qg0, kg0, vg0, lb0, wb0, mb0, idxr0)

    plsc.subcore_barrier()

    for (o, s) in _staged_rows(RPS, CH1):
        pltpu.sync_copy(acc_sp.at[pl.ds(base + o, s), :], mb0.at[pl.ds(0, s), :])
        pltpu.sync_copy(mb0.at[pl.ds(0, s), :], acc_out.at[cid, pl.ds(base + o, s), :])
        pltpu.sync_copy(s_sp.at[pl.ds(base + o, s), :], wb0.at[pl.ds(0, s), :])
        pltpu.sync_copy(wb0.at[pl.ds(0, s), :], s_out.at[cid, pl.ds(base + o, s), :])


def _sc1(q, k, v, row, col, lb16, zb, zs):
    f = functools.partial(
        pl.kernel, mesh=_mk_mesh(),
        compiler_params=pltpu.CompilerParams(use_tc_tiling_on_sc=False, needs_layout_passes=False),
        out_type=(jax.ShapeDtypeStruct((NC, N, D), jnp.float32),
                  jax.ShapeDtypeStruct((NC, N, 16), jnp.float32)),
        scratch_types=[
            pltpu.VMEM((CH1,), jnp.int32), pltpu.VMEM((CH1,), jnp.int32),
            pltpu.VMEM((CH1,), jnp.int32), pltpu.VMEM((CH1,), jnp.int32),
            pltpu.VMEM((CH1, D), jnp.float32), pltpu.VMEM((CH1, D), jnp.float32),
            pltpu.VMEM((CH1, D), jnp.float32), pltpu.VMEM((CH1, D), jnp.float32),
            pltpu.VMEM((CH1, D), jnp.float32), pltpu.VMEM((CH1, D), jnp.float32),
            pltpu.VMEM((CH1, 16), jnp.float32), pltpu.VMEM((CH1, 16), jnp.float32),
            pltpu.VMEM((CH1, 16), jnp.float32), pltpu.VMEM((CH1, 16), jnp.float32),
            pltpu.VMEM((CH1, D), jnp.float32), pltpu.VMEM((CH1, D), jnp.float32),
        ] + [pltpu.SemaphoreType.DMA] * 4 + [
            pltpu.VMEM_SHARED((N, D), jnp.float32),
            pltpu.VMEM_SHARED((N, 16), jnp.float32),
        ])(_sc1_body)
    return f(q, k, v, row, col, lb16, zb, zs)


def _sc2_body(b_hbm, c1_hbm, c2_hbm, row_hbm, col_hbm, bs_out, gs_out,
              idxr0, idxc0, idxr1, idxc1,
              b10, b20, g10, g20, b11, b21, g11, g21,
              bo0, go0, bo1, go1,
              semi0, semi1, semg0, semg1, semw0, semw1):
    w = _wid()
    nch = jnp.where(w < (NCHUNK % NW), NCHUNK // NW + 1, NCHUNK // NW)
    npairs = nch // 2

    def off(t):
        return (t * NW + w) * CH

    def issue_idx(t, idxr, idxc, semi):
        pltpu.async_copy(row_hbm.at[pl.ds(off(t), CH)], idxr, semi)
        pltpu.async_copy(col_hbm.at[pl.ds(off(t), CH)], idxc, semi)

    def wait_idx(idxr, idxc, semi):
        pltpu.make_async_copy(row_hbm.at[pl.ds(0, CH)], idxr, semi).wait()
        pltpu.make_async_copy(col_hbm.at[pl.ds(0, CH)], idxc, semi).wait()

    def issue_g(idxr, idxc, b1, b2, g1, g2, semg):
        pltpu.async_copy(b_hbm.at[idxr], b1, semg)
        pltpu.async_copy(b_hbm.at[idxc], b2, semg)
        pltpu.async_copy(c1_hbm.at[idxr], g1, semg)
        pltpu.async_copy(c2_hbm.at[idxc], g2, semg)

    def wait_g(idxr, idxc, b1, b2, g1, g2, semg):
        pltpu.make_async_copy(b_hbm.at[idxr], b1, semg).wait()
        pltpu.make_async_copy(b_hbm.at[idxc], b2, semg).wait()
        pltpu.make_async_copy(c1_hbm.at[idxr], g1, semg).wait()
        pltpu.make_async_copy(c2_hbm.at[idxc], g2, semg).wait()

    def process(t, b1, b2, g1, g2, bo, go, semw):
        @pl.when(t >= 2)
        def _():
            pltpu.make_async_copy(bo, bs_out.at[pl.ds(0, CH), :], semw).wait()
            pltpu.make_async_copy(go, gs_out.at[pl.ds(0, CH), :], semw).wait()

        def ebody(e4, carry):
            for i in range(4):
                e = e4 * 4 + i
                bo[e, :] = b1[e, :] + b2[e, :]
                for u in range(D // 16):
                    go[e, pl.ds(u * 16, 16)] = (g1[e, pl.ds(u * 16, 16)]
                                                + g2[e, pl.ds(u * 16, 16)])
            return carry

        lax.fori_loop(0, CH // 4, ebody, 0)
        pltpu.async_copy(bo, bs_out.at[pl.ds(off(t), CH), :], semw)
        pltpu.async_copy(go, gs_out.at[pl.ds(off(t), CH), :], semw)

    issue_idx(0, idxr0, idxc0, semi0)
    wait_idx(idxr0, idxc0, semi0)
    issue_g(idxr0, idxc0, b10, b20, g10, g20, semg0)
    issue_idx(1, idxr1, idxc1, semi1)

    def pair(u, carry):
        t0 = u * 2
        wait_idx(idxr1, idxc1, semi1)
        issue_g(idxr1, idxc1, b11, b21, g11, g21, semg1)
        wait_g(idxr0, idxc0, b10, b20, g10, g20, semg0)
        process(t0, b10, b20, g10, g20, bo0, go0, semw0)

        @pl.when(t0 + 2 < nch)
        def _():
            issue_idx(t0 + 2, idxr0, idxc0, semi0)

        wait_g(idxr1, idxc1, b11, b21, g11, g21, semg1)
        process(t0 + 1, b11, b21, g11, g21, bo1, go1, semw1)

        @pl.when(t0 + 2 < nch)
        def _():
            wait_idx(idxr0, idxc0, semi0)
            issue_g(idxr0, idxc0, b10, b20, g10, g20, semg0)

        @pl.when(t0 + 3 < nch)
        def _():
            issue_idx(t0 + 3, idxr1, idxc1, semi1)

        return carry

    lax.fori_loop(0, npairs, pair, 0)

    @pl.when(nch % 2 == 1)
    def _():
        wait_g(idxr0, idxc0, b10, b20, g10, g20, semg0)
        process(npairs * 2, b10, b20, g10, g20, bo0, go0, semw0)

    pltpu.make_async_copy(bo0, bs_out.at[pl.ds(0, CH), :], semw0).wait()
    pltpu.make_async_copy(go0, gs_out.at[pl.ds(0, CH), :], semw0).wait()
    pltpu.make_async_copy(bo1, bs_out.at[pl.ds(0, CH), :], semw1).wait()
    pltpu.make_async_copy(go1, gs_out.at[pl.ds(0, CH), :], semw1).wait()


def _sc2(b_tab, c1, c2, row, col):
    f = functools.partial(
        pl.kernel, mesh=_mk_mesh(),
        compiler_params=pltpu.CompilerParams(use_tc_tiling_on_sc=False, needs_layout_passes=False),
        out_type=(jax.ShapeDtypeStruct((E, 16), jnp.float32),
                  jax.ShapeDtypeStruct((E, D), jnp.float32)),
        scratch_types=[
            pltpu.VMEM((CH,), jnp.int32), pltpu.VMEM((CH,), jnp.int32),
            pltpu.VMEM((CH,), jnp.int32), pltpu.VMEM((CH,), jnp.int32),
            pltpu.VMEM((CH, 16), jnp.float32), pltpu.VMEM((CH, 16), jnp.float32),
            pltpu.VMEM((CH, D), jnp.float32), pltpu.VMEM((CH, D), jnp.float32),
            pltpu.VMEM((CH, 16), jnp.float32), pltpu.VMEM((CH, 16), jnp.float32),
            pltpu.VMEM((CH, D), jnp.float32), pltpu.VMEM((CH, D), jnp.float32),
            pltpu.VMEM((CH, 16), jnp.float32), pltpu.VMEM((CH, D), jnp.float32),
            pltpu.VMEM((CH, 16), jnp.float32), pltpu.VMEM((CH, D), jnp.float32),
        ] + [pltpu.SemaphoreType.DMA] * 6)(_sc2_body)
    return f(b_tab, c1, c2, row, col)


def _sc3_body(con_hbm, row_hbm, zs_hbm, agg_out,
              idxr0, idxr1, cb0, cb1, sb, semi0, semi1, agg_sp):
    cid = lax.axis_index("c")
    sid = lax.axis_index("s")
    w = sid * NC + cid
    base = sid * RPS
    nch = jnp.where(w < (NCHUNK % NW), NCHUNK // NW + 1, NCHUNK // NW)
    npairs = nch // 2

    pltpu.sync_copy(zs_hbm, sb)
    pltpu.sync_copy(sb, agg_sp.at[pl.ds(base, RPS), :])
    plsc.subcore_barrier()

    def off(t):
        return (t * NW + w) * CH

    def issue(t, idxr, cb, semi):
        pltpu.async_copy(row_hbm.at[pl.ds(off(t), CH)], idxr, semi)
        pltpu.async_copy(con_hbm.at[pl.ds(off(t), CH), :], cb, semi)

    def wait(idxr, cb, semi):
        pltpu.make_async_copy(row_hbm.at[pl.ds(0, CH)], idxr, semi).wait()
        pltpu.make_async_copy(con_hbm.at[pl.ds(0, CH), :], cb, semi).wait()

    issue(0, idxr0, cb0, semi0)
    issue(1, idxr1, cb1, semi1)

    def pair(u, carry):
        t0 = u * 2
        wait(idxr0, cb0, semi0)
        pltpu.sync_copy(cb0, agg_sp.at[idxr0], add=True)

        @pl.when(t0 + 2 < nch)
        def _():
            issue(t0 + 2, idxr0, cb0, semi0)

        wait(idxr1, cb1, semi1)
        pltpu.sync_copy(cb1, agg_sp.at[idxr1], add=True)

        @pl.when(t0 + 3 < nch)
        def _():
            issue(t0 + 3, idxr1, cb1, semi1)

        return carry

    lax.fori_loop(0, npairs, pair, 0)

    @pl.when(nch % 2 == 1)
    def _():
        wait(idxr0, cb0, semi0)
        pltpu.sync_copy(cb0, agg_sp.at[idxr0], add=True)

    plsc.subcore_barrier()

    pltpu.sync_copy(agg_sp.at[pl.ds(base, RPS), :], sb)
    pltpu.sync_copy(sb, agg_out.at[cid, pl.ds(base, RPS), :])


def _sc3(contrib, row, zs):
    f = functools.partial(
        pl.kernel, mesh=_mk_mesh(),
        compiler_params=pltpu.CompilerParams(use_tc_tiling_on_sc=False, needs_layout_passes=False),
        out_type=jax.ShapeDtypeStruct((NC, N, 16), jnp.float32),
        scratch_types=[
            pltpu.VMEM((CH,), jnp.int32), pltpu.VMEM((CH,), jnp.int32),
            pltpu.VMEM((CH, 16), jnp.float32), pltpu.VMEM((CH, 16), jnp.float32),
            pltpu.VMEM((RPS, 16), jnp.float32),
        ] + [pltpu.SemaphoreType.DMA] * 2 + [
            pltpu.VMEM_SHARED((N, 16), jnp.float32),
        ])(_sc3_body)
    return f(contrib, row, zs)


def _ln(x, eps=1e-6):
    m = jnp.mean(x, axis=-1, keepdims=True)
    v = jnp.mean((x - m) * (x - m), axis=-1, keepdims=True)
    return (x - m) / jnp.sqrt(v + eps)


def _silu(x):
    return x / (1.0 + jnp.exp(-x))


def _mod(x, sh, sc):
    return x * (1 + sc) + sh


def _dot(a, b):
    return jax.lax.dot_general(a, b, (((1,), (0,)), ((), ())),
                               preferred_element_type=jnp.float32)


def _bcast_spec(arr):
    nd = arr.ndim
    return pl.BlockSpec(arr.shape, lambda i: (0,) * nd)


def _rows_spec(blk, cols):
    return pl.BlockSpec((blk, cols), lambda i: (i, 0))


_TC_PARAMS = pltpu.CompilerParams(dimension_semantics=("arbitrary",))


def _tca_body(h, nte, wtm, btm, wq, wk, wv, q_o, k_o, v_o,
              g1_o, g2_o, g3_o, g4_o):
    tm = _dot(_silu(nte[...]), wtm[...]) + btm[...]
    hm = _mod(_ln(h[...]), tm[:, 0:D], tm[:, D:2 * D])
    q_o[...] = _dot(hm, wq[...])
    k_o[...] = _dot(hm, wk[...])
    v_o[...] = _dot(hm, wv[...])
    g1_o[...] = tm[:, 2 * D:3 * D]
    g2_o[...] = tm[:, 3 * D:4 * D]
    g3_o[...] = tm[:, 4 * D:5 * D]
    g4_o[...] = tm[:, 5 * D:6 * D]


def _tca(h, nte, p):
    blk = 1000
    wtm, btm = p['node_tm_W'], p['node_tm_b'].reshape(1, -1)
    outs = [jax.ShapeDtypeStruct((N, D), jnp.float32)] * 7
    return pl.pallas_call(
        _tca_body,
        grid=(N // blk,),
        in_specs=[_rows_spec(blk, D), _rows_spec(blk, D),
                  _bcast_spec(wtm), _bcast_spec(btm),
                  _bcast_spec(p['Wq']), _bcast_spec(p['Wk']), _bcast_spec(p['Wv'])],
        out_specs=[_rows_spec(blk, D)] * 7,
        out_shape=outs,
        compiler_params=_TC_PARAMS,
    )(h, nte, wtm, btm, p['Wq'], p['Wk'], p['Wv'])


def _tcb_body(pdiff, ea, ete, ex8, wd, we2, be2, wtm, btm, wl, wx8,
              misc_o, g1_o, g2_o, g3_o, g4_o):
    pd = pdiff[...]
    dist = jnp.sum(pd * pd, axis=-1, keepdims=True)
    e2 = dist * wd[...] + _dot(ea[...], we2[...]) + be2[...]
    tm = _dot(_silu(ete[...]), wtm[...]) + btm[...]
    emod = _mod(_ln(e2), tm[:, 0:16], tm[:, 16:32])
    lb = _dot(emod, wl[...]) + _dot(ex8[...], wx8[...])
    blk = lb.shape[0]
    misc_o[...] = jnp.concatenate(
        [lb, dist, jnp.zeros((blk, 7), jnp.float32)], axis=1)
    g1_o[...] = tm[:, 32:48]
    g2_o[...] = tm[:, 48:64]
    g3_o[...] = tm[:, 64:80]
    g4_o[...] = tm[:, 80:96]


def _tcb(pdiff, ea, ete, ex8, p):
    blk = 2000
    wd = p['edge_emb_W'][0:1]
    we2 = p['edge_emb_W'][1:]
    be2 = p['edge_emb_b'].reshape(1, -1)
    wtm, btm = p['edge_tm_W'], p['edge_tm_b'].reshape(1, -1)
    wx8 = jnp.pad(p['Wx'], ((0, 6), (0, 0)))
    outs = [jax.ShapeDtypeStruct((E, 16), jnp.float32)] * 5
    return pl.pallas_call(
        _tcb_body,
        grid=(E // blk,),
        in_specs=[_rows_spec(blk, 16), _rows_spec(blk, 16),
                  _rows_spec(blk, D), _rows_spec(blk, 8),
                  _bcast_spec(wd), _bcast_spec(we2), _bcast_spec(be2),
                  _bcast_spec(wtm), _bcast_spec(btm),
                  _bcast_spec(p['We']), _bcast_spec(wx8)],
        out_specs=[_rows_spec(blk, 16)] * 5,
        out_shape=outs,
        compiler_params=_TC_PARAMS,
    )(pdiff, ea, ete, ex8, wd, we2, be2, wtm, btm, p['We'], wx8)


def _tcc_body(acc3, s3, h, nmask, g1, g2, g3, g4, sexp, wo, n2e,
              ff1, fb1, ff2, fb2, eqw1, eqw2,
              hout_o, b_o, c1_o, c2_o):
    acc = acc3[0] + acc3[1]
    sv = (s3[0] + s3[1])[:, 0:8]
    se = jax.lax.dot_general(sv, sexp[...], (((1,), (0,)), ((), ())),
                             precision=jax.lax.Precision.HIGHEST,
                             preferred_element_type=jnp.float32)
    att = _dot(acc / (se + 1e-16), wo[...])
    b_o[...] = _dot(att, n2e[...])
    h_node = h[...] + g1[...] * att
    hml = _mod(_ln(h_node), g2[...], g3[...]) * nmask[...]
    ffn = _dot(_silu(_dot(hml, ff1[...]) + fb1[...]), ff2[...]) + fb2[...]
    h_out = (hml + g4[...] * ffn) * nmask[...]
    hout_o[...] = h_out
    c1_o[...] = _dot(h_out, eqw1[...])
    c2_o[...] = _dot(h_out, eqw2[...])


def _tcc(acc_p, s_p, h, nmask, g1, g2, g3, g4, p):
    blk = 1000
    sexp = jnp.kron(jnp.eye(8, dtype=jnp.float32),
                    jnp.ones((1, DH), jnp.float32))
    fb1 = p['ff1_b'].reshape(1, -1)
    fb2 = p['ff2_b'].reshape(1, -1)
    eqw1 = p['eq_in_W'][0:D]
    eqw2 = p['eq_in_W'][D:2 * D]
    outs = [jax.ShapeDtypeStruct((N, D), jnp.float32),
            jax.ShapeDtypeStruct((N, 16), jnp.float32),
            jax.ShapeDtypeStruct((N, D), jnp.float32),
            jax.ShapeDtypeStruct((N, D), jnp.float32)]
    return pl.pallas_call(
        _tcc_body,
        grid=(N // blk,),
        in_specs=[pl.BlockSpec((NC, blk, D), lambda i: (0, i, 0)),
                  pl.BlockSpec((NC, blk, 16), lambda i: (0, i, 0)),
                  _rows_spec(blk, D), _rows_spec(blk, 1),
                  _rows_spec(blk, D), _rows_spec(blk, D),
                  _rows_spec(blk, D), _rows_spec(blk, D),
                  _bcast_spec(sexp), _bcast_spec(p['Wo']), _bcast_spec(p['n2e_W']),
                  _bcast_spec(p['ff1_W']), _bcast_spec(fb1),
                  _bcast_spec(p['ff2_W']), _bcast_spec(fb2),
                  _bcast_spec(eqw1), _bcast_spec(eqw2)],
        out_specs=[_rows_spec(blk, D), _rows_spec(blk, 16),
                   _rows_spec(blk, D), _rows_spec(blk, D)],
        out_shape=outs,
        compiler_params=_TC_PARAMS,
    )(acc_p, s_p, h, nmask, g1, g2, g3, g4, sexp, p['Wo'], p['n2e_W'],
      p['ff1_W'], fb1, p['ff2_W'], fb2, eqw1, eqw2)


def _tcde_body(ea, g1, g2, g3, g4, bsum, gsum, misc, pdiff, ete, ex8,
               n2eb, ff3, fb3, ff4, fb4, wtm, btm, w3, w4, eqb,
               c1w, c1b, c2w, cscale,
               heo_o, con_o):
    he = ea[...] + g1[...] * (bsum[...] + n2eb[...])
    he = _mod(_ln(he), g2[...], g3[...])
    ffe = _dot(_silu(_dot(he, ff3[...]) + fb3[...]), ff4[...]) + fb4[...]
    heo = he + g4[...] * ffe
    heo_o[...] = heo
    tm = _dot(_silu(ete[...]), wtm[...]) + btm[...]
    dist = misc[:, 8:9]
    lin = gsum[...] + _dot(heo, w3[...]) + dist * w4[...] + eqb[...]
    inv = _mod(_ln(lin), tm[:, 0:D], tm[:, D:2 * D])
    u = jnp.tanh(_dot(_silu(_dot(inv, c1w[...]) + c1b[...]), c2w[...]))
    blk = u.shape[0]
    adjs = jnp.concatenate([jnp.ones((blk, 1), jnp.float32), ex8[:, 0:7]],
                           axis=1)
    invm = jnp.sum(u * adjs, axis=-1, keepdims=True) * (1.0 / 3.0)
    nrm = jnp.sqrt(dist)
    cdf = pdiff[...] / jnp.maximum(nrm, 1e-8) * cscale[...]
    con_o[...] = cdf * invm


def _tcde(ea, g1, g2, g3, g4, bsum, gsum, misc, pdiff, ete, ex8, p):
    blk = 2000
    n2eb = p['n2e_b'].reshape(1, -1)
    fb3 = p['ff3_b'].reshape(1, -1)
    fb4 = p['ff4_b'].reshape(1, -1)
    wtm, btm = p['eq_tm_W'], p['eq_tm_b'].reshape(1, -1)
    w3 = p['eq_in_W'][2 * D:2 * D + 16]
    w4 = p['eq_in_W'][2 * D + 16:2 * D + 17]
    eqb = p['eq_in_b'].reshape(1, -1)
    c1b = p['eq_c1_b'].reshape(1, -1)
    c2w = jnp.pad(p['eq_c2_W'], ((0, 0), (0, 5)))
    cscale = p['coors_scale'].reshape(1, 1)
    outs = [jax.ShapeDtypeStruct((E, 16), jnp.float32),
            jax.ShapeDtypeStruct((E, 16), jnp.float32)]
    return pl.pallas_call(
        _tcde_body,
        grid=(E // blk,),
        in_specs=[_rows_spec(blk, 16), _rows_spec(blk, 16), _rows_spec(blk, 16),
                  _rows_spec(blk, 16), _rows_spec(blk, 16), _rows_spec(blk, 16),
                  _rows_spec(blk, D), _rows_spec(blk, 16), _rows_spec(blk, 16),
                  _rows_spec(blk, D), _rows_spec(blk, 8),
                  _bcast_spec(n2eb), _bcast_spec(p['ff3_W']), _bcast_spec(fb3),
                  _bcast_spec(p['ff4_W']), _bcast_spec(fb4),
                  _bcast_spec(wtm), _bcast_spec(btm), _bcast_spec(w3),
                  _bcast_spec(w4), _bcast_spec(eqb),
                  _bcast_spec(p['eq_c1_W']), _bcast_spec(c1b), _bcast_spec(c2w),
                  _bcast_spec(cscale)],
        out_specs=[_rows_spec(blk, 16), _rows_spec(blk, 16)],
        out_shape=outs,
        compiler_params=_TC_PARAMS,
    )(ea, g1, g2, g3, g4, bsum, gsum, misc, pdiff, ete, ex8,
      n2eb, p['ff3_W'], fb3, p['ff4_W'], fb4, wtm, btm, w3, w4, eqb,
      p['eq_c1_W'], c1b, c2w, cscale)


def _tcf_body(pos16, agg3, out_o):
    out_o[...] = pos16[...] + agg3[0] + agg3[1]


def _tcf(pos16, agg_p):
    blk = 1000
    return pl.pallas_call(
        _tcf_body,
        grid=(N // blk,),
        in_specs=[_rows_spec(blk, 16),
                  pl.BlockSpec((NC, blk, 16), lambda i: (0, i, 0))],
        out_specs=_rows_spec(blk, 16),
        out_shape=jax.ShapeDtypeStruct((N, 16), jnp.float32),
        compiler_params=_TC_PARAMS,
    )(pos16, agg_p)


def kernel(pos, h, edge_attr, edge_index, node_mask, extra_heads,
           node_time_emb, edge_time_emb, params):
    p = params
    row = edge_index[0]
    col = edge_index[1]
    pos16 = jnp.pad(pos, ((0, 0), (0, 13)))
    ex8 = jnp.pad(extra_heads, ((0, 0), (0, 6)))
    zb = jnp.zeros((CH1, D), jnp.float32)
    zs1 = jnp.zeros((CH1, 16), jnp.float32)
    zs3 = jnp.zeros((RPS, 16), jnp.float32)

    pdiff = _sc0(pos16, row, col)
    q, k, v, g1, g2, g3, g4 = _tca(h, node_time_emb, p)
    misc, e1, e2m, e3, e4 = _tcb(pdiff, edge_attr, edge_time_emb, ex8, p)
    acc_p, s_p = _sc1(q, k, v, row, col, misc, zb, zs1)
    h_out, b_tab, c1, c2 = _tcc(acc_p, s_p, h, node_mask, g1, g2, g3, g4, p)
    bsum, gsum = _sc2(b_tab, c1, c2, row, col)
    h_edge_out, contrib = _tcde(edge_attr, e1, e2m, e3, e4, bsum, gsum,
                                misc, pdiff, edge_time_emb, ex8, p)
    agg_p = _sc3(contrib, row, zs3)
    pos_out16 = _tcf(pos16, agg_p)
    return h_out, h_edge_out, pos_out16[:, 0:3]


# speedup vs baseline: 1.8798x; 1.1033x over previous
import functools

import jax
import jax.numpy as jnp
import numpy as np
from jax import lax
from jax.experimental import pallas as pl
from jax.experimental.pallas import tpu as pltpu
from jax.experimental.pallas import tpu_sc as plsc

N = 10000
E = 160000
D = 128
ED = 16
NH = 8
DH = 16
NC = 2
NS = 16
NW = NC * NS
CH = 128
NCHUNK = E // CH
RPS = N // NS
INV_SQRT_DH = 1.0 / np.sqrt(DH)

def _mk_mesh():
    return plsc.VectorSubcoreMesh(core_axis_name="c", subcore_axis_name="s",
                                  num_cores=NC, num_subcores=NS)


def _wid():
    return lax.axis_index("s") * NC + lax.axis_index("c")


def _chunk_loop(body, nchunk=NCHUNK):
    w = _wid()
    nch = jnp.where(w < (nchunk % NW), nchunk // NW + 1, nchunk // NW)

    def tbody(t, carry):
        body(t * NW + w)
        return carry

    lax.fori_loop(0, nch, tbody, 0)


def _staged_rows(n, step=64):
    out = []
    o = 0
    while o < n:
        s = min(step, n - o)
        out.append((o, s))
        o += s
    return out


def _sc0_body(pos_hbm, row_hbm, col_hbm, out_hbm,
              idxr0, idxc0, idxr1, idxc1, pa0, pb0, pa1, pb1, ob0, ob1,
              semi0, semi1, semg0, semg1, semw0, semw1):
    w = _wid()
    nch = jnp.where(w < (NCHUNK % NW), NCHUNK // NW + 1, NCHUNK // NW)
    npairs = nch // 2

    def off(t):
        return (t * NW + w) * CH

    def issue_idx(t, idxr, idxc, semi):
        pltpu.async_copy(row_hbm.at[pl.ds(off(t), CH)], idxr, semi)
        pltpu.async_copy(col_hbm.at[pl.ds(off(t), CH)], idxc, semi)

    def wait_idx(idxr, idxc, semi):
        pltpu.make_async_copy(row_hbm.at[pl.ds(0, CH)], idxr, semi).wait()
        pltpu.make_async_copy(col_hbm.at[pl.ds(0, CH)], idxc, semi).wait()

    def issue_g(idxr, idxc, pa, pb, semg):
        pltpu.async_copy(pos_hbm.at[idxr], pa, semg)
        pltpu.async_copy(pos_hbm.at[idxc], pb, semg)

    def wait_g(idxr, idxc, pa, pb, semg):
        pltpu.make_async_copy(pos_hbm.at[idxr], pa, semg).wait()
        pltpu.make_async_copy(pos_hbm.at[idxc], pb, semg).wait()

    def process(t, pa, pb, ob, semw):
        @pl.when(t >= 2)
        def _():
            pltpu.make_async_copy(ob, out_hbm.at[pl.ds(0, CH), :], semw).wait()

        def ebody(e4, carry):
            for i in range(4):
                e = e4 * 4 + i
                ob[e, :] = pa[e, :] - pb[e, :]
            return carry

        lax.fori_loop(0, CH // 4, ebody, 0)
        pltpu.async_copy(ob, out_hbm.at[pl.ds(off(t), CH), :], semw)

    issue_idx(0, idxr0, idxc0, semi0)
    wait_idx(idxr0, idxc0, semi0)
    issue_g(idxr0, idxc0, pa0, pb0, semg0)
    issue_idx(1, idxr1, idxc1, semi1)

    def pair(u, carry):
        t0 = u * 2
        wait_idx(idxr1, idxc1, semi1)
        issue_g(idxr1, idxc1, pa1, pb1, semg1)
        wait_g(idxr0, idxc0, pa0, pb0, semg0)
        process(t0, pa0, pb0, ob0, semw0)

        @pl.when(t0 + 2 < nch)
        def _():
            issue_idx(t0 + 2, idxr0, idxc0, semi0)

        wait_g(idxr1, idxc1, pa1, pb1, semg1)
        process(t0 + 1, pa1, pb1, ob1, semw1)

        @pl.when(t0 + 2 < nch)
        def _():
            wait_idx(idxr0, idxc0, semi0)
            issue_g(idxr0, idxc0, pa0, pb0, semg0)

        @pl.when(t0 + 3 < nch)
        def _():
            issue_idx(t0 + 3, idxr1, idxc1, semi1)

        return carry

    lax.fori_loop(0, npairs, pair, 0)

    @pl.when(nch % 2 == 1)
    def _():
        wait_g(idxr0, idxc0, pa0, pb0, semg0)
        process(npairs * 2, pa0, pb0, ob0, semw0)

    pltpu.make_async_copy(ob0, out_hbm.at[pl.ds(0, CH), :], semw0).wait()
    pltpu.make_async_copy(ob1, out_hbm.at[pl.ds(0, CH), :], semw1).wait()


def _sc0(pos16, row, col):
    f = functools.partial(
        pl.kernel, mesh=_mk_mesh(),
        compiler_params=pltpu.CompilerParams(use_tc_tiling_on_sc=False, needs_layout_passes=False),
        out_type=jax.ShapeDtypeStruct((E, 16), jnp.float32),
        scratch_types=[
            pltpu.VMEM((CH,), jnp.int32), pltpu.VMEM((CH,), jnp.int32),
            pltpu.VMEM((CH,), jnp.int32), pltpu.VMEM((CH,), jnp.int32),
            pltpu.VMEM((CH, 16), jnp.float32), pltpu.VMEM((CH, 16), jnp.float32),
            pltpu.VMEM((CH, 16), jnp.float32), pltpu.VMEM((CH, 16), jnp.float32),
            pltpu.VMEM((CH, 16), jnp.float32), pltpu.VMEM((CH, 16), jnp.float32),
        ] + [pltpu.SemaphoreType.DMA] * 6)(_sc0_body)
    return f(pos16, row, col)


CH1 = 32
NCHUNK1 = E // CH1


def _sc1_body(q_hbm, k_hbm, v_hbm, row_hbm, col_hbm, lb_hbm, zb_hbm, zs_hbm,
              acc_out, s_out,
              idxr0, idxc0, idxr1, idxc1,
              qg0, kg0, vg0, qg1, kg1, vg1,
              lb0, lb1, wb0, wb1, mb0, mb1,
              semi0, semi1, semg0, semg1,
              acc_sp, s_sp):
    cid = lax.axis_index("c")
    sid = lax.axis_index("s")
    w = sid * NC + cid
    base = sid * RPS
    nch = jnp.where(w < (NCHUNK1 % NW), NCHUNK1 // NW + 1, NCHUNK1 // NW)
    npairs = nch // 2

    pltpu.sync_copy(zb_hbm, mb0)
    pltpu.sync_copy(zs_hbm, wb0)
    for (o, s) in _staged_rows(RPS, CH1):
        pltpu.sync_copy(mb0.at[pl.ds(0, s), :], acc_sp.at[pl.ds(base + o, s), :])
        pltpu.sync_copy(wb0.at[pl.ds(0, s), :], s_sp.at[pl.ds(base + o, s), :])
    plsc.subcore_barrier()

    lane = lax.broadcasted_iota(jnp.int32, (16,), 0)

    def off(t):
        return (t * NW + w) * CH1

    def issue_idx(t, idxr, idxc, semi):
        pltpu.async_copy(row_hbm.at[pl.ds(off(t), CH1)], idxr, semi)
        pltpu.async_copy(col_hbm.at[pl.ds(off(t), CH1)], idxc, semi)

    def wait_idx(idxr, idxc, semi):
        pltpu.make_async_copy(row_hbm.at[pl.ds(0, CH1)], idxr, semi).wait()
        pltpu.make_async_copy(col_hbm.at[pl.ds(0, CH1)], idxc, semi).wait()

    def issue_g(t, idxr, idxc, qg, kg, vg, lb, semg):
        pltpu.async_copy(q_hbm.at[idxr], qg, semg)
        pltpu.async_copy(k_hbm.at[idxc], kg, semg)
        pltpu.async_copy(v_hbm.at[idxc], vg, semg)
        pltpu.async_copy(lb_hbm.at[pl.ds(off(t), CH1), :], lb, semg)

    def wait_g(idxr, idxc, qg, kg, vg, lb, semg):
        pltpu.make_async_copy(q_hbm.at[idxr], qg, semg).wait()
        pltpu.make_async_copy(k_hbm.at[idxc], kg, semg).wait()
        pltpu.make_async_copy(v_hbm.at[idxc], vg, semg).wait()
        pltpu.make_async_copy(lb_hbm.at[pl.ds(0, CH1), :], lb, semg).wait()

    def process(qg, kg, vg, lb, wb, mb, idxr):
        def ebody(e, carry):
            lvec = jnp.zeros((16,), jnp.float32)
            for h in range(NH):
                dh = jnp.sum(qg[e, pl.ds(h * DH, DH)] * kg[e, pl.ds(h * DH, DH)])
                lvec = jnp.where(lane == h, dh, lvec)
            wv = jnp.exp(lvec * INV_SQRT_DH + lb[e, :])
            wb[e, :] = wv
            for h in range(NH):
                mb[e, pl.ds(h * DH, DH)] = wv[h] * vg[e, pl.ds(h * DH, DH)]
            return carry

        lax.fori_loop(0, CH1, ebody, 0)
        pltpu.sync_copy(mb, acc_sp.at[idxr], add=True)
        pltpu.sync_copy(wb, s_sp.at[idxr], add=True)

    issue_idx(0, idxr0, idxc0, semi0)
    wait_idx(idxr0, idxc0, semi0)
    issue_g(0, idxr0, idxc0, qg0, kg0, vg0, lb0, semg0)
    issue_idx(1, idxr1, idxc1, semi1)

    def pair(u, carry):
        t0 = u * 2
        wait_idx(idxr1, idxc1, semi1)
        issue_g(t0 + 1, idxr1, idxc1, qg1, kg1, vg1, lb1, semg1)
        wait_g(idxr0, idxc0, qg0, kg0, vg0, lb0, semg0)
        process(qg0, kg0, vg0, lb0, wb0, mb0, idxr0)

        @pl.when(t0 + 2 < nch)
        def _():
            issue_idx(t0 + 2, idxr0, idxc0, semi0)

        wait_g(idxr1, idxc1, qg1, kg1, vg1, lb1, semg1)
        process(qg1, kg1, vg1, lb1, wb1, mb1, idxr1)

        @pl.when(t0 + 2 < nch)
        def _():
            wait_idx(idxr0, idxc0, semi0)
            issue_g(t0 + 2, idxr0, idxc0, qg0, kg0, vg0, lb0, semg0)

        @pl.when(t0 + 3 < nch)
        def _():
            issue_idx(t0 + 3, idxr1, idxc1, semi1)

        return carry

    lax.fori_loop(0, npairs, pair, 0)

    @pl.when(nch % 2 == 1)
    def _():
        wait_g(idxr0, idxc0, qg0, kg0, vg0, lb0, semg0)
        process(qg0, kg0, vg0, lb0, wb0, mb0, idxr0)

    plsc.subcore_barrier()

    for (o, s) in _staged_rows(RPS, CH1):
        pltpu.sync_copy(acc_sp.at[pl.ds(base + o, s), :], mb0.at[pl.ds(0, s), :])
        pltpu.sync_copy(mb0.at[pl.ds(0, s), :], acc_out.at[cid, pl.ds(base + o, s), :])
        pltpu.sync_copy(s_sp.at[pl.ds(base + o, s), :], wb0.at[pl.ds(0, s), :])
        pltpu.sync_copy(wb0.at[pl.ds(0, s), :], s_out.at[cid, pl.ds(base + o, s), :])


def _sc1(q, k, v, row, col, lb16, zb, zs):
    f = functools.partial(
        pl.kernel, mesh=_mk_mesh(),
        compiler_params=pltpu.CompilerParams(use_tc_tiling_on_sc=False, needs_layout_passes=False),
        out_type=(jax.ShapeDtypeStruct((NC, N, D), jnp.float32),
                  jax.ShapeDtypeStruct((NC, N, 16), jnp.float32)),
        scratch_types=[
            pltpu.VMEM((CH1,), jnp.int32), pltpu.VMEM((CH1,), jnp.int32),
            pltpu.VMEM((CH1,), jnp.int32), pltpu.VMEM((CH1,), jnp.int32),
            pltpu.VMEM((CH1, D), jnp.float32), pltpu.VMEM((CH1, D), jnp.float32),
            pltpu.VMEM((CH1, D), jnp.float32), pltpu.VMEM((CH1, D), jnp.float32),
            pltpu.VMEM((CH1, D), jnp.float32), pltpu.VMEM((CH1, D), jnp.float32),
            pltpu.VMEM((CH1, 16), jnp.float32), pltpu.VMEM((CH1, 16), jnp.float32),
            pltpu.VMEM((CH1, 16), jnp.float32), pltpu.VMEM((CH1, 16), jnp.float32),
            pltpu.VMEM((CH1, D), jnp.float32), pltpu.VMEM((CH1, D), jnp.float32),
        ] + [pltpu.SemaphoreType.DMA] * 4 + [
            pltpu.VMEM_SHARED((N, D), jnp.float32),
            pltpu.VMEM_SHARED((N, 16), jnp.float32),
        ])(_sc1_body)
    return f(q, k, v, row, col, lb16, zb, zs)


def _sc2_body(b_hbm, c1_hbm, c2_hbm, row_hbm, col_hbm,
              bs_out, g1_out, g2_out,
              idxr0, idxc0, idxr1, idxc1,
              b10, b20, g10, g20, b11, b21, g11, g21,
              bo0, bo1,
              semi0, semi1, semg0, semg1, semw0, semw1, semx0, semx1):
    w = _wid()
    nch = jnp.where(w < (NCHUNK % NW), NCHUNK // NW + 1, NCHUNK // NW)
    npairs = nch // 2

    def off(t):
        return (t * NW + w) * CH

    def issue_idx(t, idxr, idxc, semi):
        pltpu.async_copy(row_hbm.at[pl.ds(off(t), CH)], idxr, semi)
        pltpu.async_copy(col_hbm.at[pl.ds(off(t), CH)], idxc, semi)

    def wait_idx(idxr, idxc, semi):
        pltpu.make_async_copy(row_hbm.at[pl.ds(0, CH)], idxr, semi).wait()
        pltpu.make_async_copy(col_hbm.at[pl.ds(0, CH)], idxc, semi).wait()

    def issue_g(idxr, idxc, b1, b2, g1, g2, semg):
        pltpu.async_copy(b_hbm.at[idxr], b1, semg)
        pltpu.async_copy(b_hbm.at[idxc], b2, semg)
        pltpu.async_copy(c1_hbm.at[idxr], g1, semg)
        pltpu.async_copy(c2_hbm.at[idxc], g2, semg)

    def wait_g(idxr, idxc, b1, b2, g1, g2, semg):
        pltpu.make_async_copy(b_hbm.at[idxr], b1, semg).wait()
        pltpu.make_async_copy(b_hbm.at[idxc], b2, semg).wait()
        pltpu.make_async_copy(c1_hbm.at[idxr], g1, semg).wait()
        pltpu.make_async_copy(c2_hbm.at[idxc], g2, semg).wait()

    def wait_gw(g1, g2, semx):
        pltpu.make_async_copy(g1, g1_out.at[pl.ds(0, CH), :], semx).wait()
        pltpu.make_async_copy(g2, g2_out.at[pl.ds(0, CH), :], semx).wait()

    def process(t, b1, b2, g1, g2, bo, semw, semx):
        pltpu.async_copy(g1, g1_out.at[pl.ds(off(t), CH), :], semx)
        pltpu.async_copy(g2, g2_out.at[pl.ds(off(t), CH), :], semx)

        @pl.when(t >= 2)
        def _():
            pltpu.make_async_copy(bo, bs_out.at[pl.ds(0, CH), :], semw).wait()

        def ebody(e4, carry):
            for i in range(4):
                e = e4 * 4 + i
                bo[e, :] = b1[e, :] + b2[e, :]
            return carry

        lax.fori_loop(0, CH // 4, ebody, 0)
        pltpu.async_copy(bo, bs_out.at[pl.ds(off(t), CH), :], semw)

    issue_idx(0, idxr0, idxc0, semi0)
    wait_idx(idxr0, idxc0, semi0)
    issue_g(idxr0, idxc0, b10, b20, g10, g20, semg0)
    issue_idx(1, idxr1, idxc1, semi1)

    def pair(u, carry):
        t0 = u * 2
        wait_idx(idxr1, idxc1, semi1)

        @pl.when(t0 > 0)
        def _():
            wait_gw(g11, g21, semx1)

        issue_g(idxr1, idxc1, b11, b21, g11, g21, semg1)
        wait_g(idxr0, idxc0, b10, b20, g10, g20, semg0)
        process(t0, b10, b20, g10, g20, bo0, semw0, semx0)

        @pl.when(t0 + 2 < nch)
        def _():
            issue_idx(t0 + 2, idxr0, idxc0, semi0)

        wait_g(idxr1, idxc1, b11, b21, g11, g21, semg1)
        process(t0 + 1, b11, b21, g11, g21, bo1, semw1, semx1)

        @pl.when(t0 + 2 < nch)
        def _():
            wait_idx(idxr0, idxc0, semi0)
            wait_gw(g10, g20, semx0)
            issue_g(idxr0, idxc0, b10, b20, g10, g20, semg0)

        @pl.when(t0 + 3 < nch)
        def _():
            issue_idx(t0 + 3, idxr1, idxc1, semi1)

        return carry

    lax.fori_loop(0, npairs, pair, 0)

    @pl.when(nch % 2 == 1)
    def _():
        wait_g(idxr0, idxc0, b10, b20, g10, g20, semg0)
        process(npairs * 2, b10, b20, g10, g20, bo0, semw0, semx0)

    wait_gw(g10, g20, semx0)
    wait_gw(g11, g21, semx1)
    pltpu.make_async_copy(bo0, bs_out.at[pl.ds(0, CH), :], semw0).wait()
    pltpu.make_async_copy(bo1, bs_out.at[pl.ds(0, CH), :], semw1).wait()


def _sc2(b_tab, c1, c2, row, col):
    f = functools.partial(
        pl.kernel, mesh=_mk_mesh(),
        compiler_params=pltpu.CompilerParams(use_tc_tiling_on_sc=False, needs_layout_passes=False),
        out_type=(jax.ShapeDtypeStruct((E, 16), jnp.float32),
                  jax.ShapeDtypeStruct((E, D), jnp.float32),
                  jax.ShapeDtypeStruct((E, D), jnp.float32)),
        scratch_types=[
            pltpu.VMEM((CH,), jnp.int32), pltpu.VMEM((CH,), jnp.int32),
            pltpu.VMEM((CH,), jnp.int32), pltpu.VMEM((CH,), jnp.int32),
            pltpu.VMEM((CH, 16), jnp.float32), pltpu.VMEM((CH, 16), jnp.float32),
            pltpu.VMEM((CH, D), jnp.float32), pltpu.VMEM((CH, D), jnp.float32),
            pltpu.VMEM((CH, 16), jnp.float32), pltpu.VMEM((CH, 16), jnp.float32),
            pltpu.VMEM((CH, D), jnp.float32), pltpu.VMEM((CH, D), jnp.float32),
            pltpu.VMEM((CH, 16), jnp.float32), pltpu.VMEM((CH, 16), jnp.float32),
        ] + [pltpu.SemaphoreType.DMA] * 8)(_sc2_body)
    return f(b_tab, c1, c2, row, col)


def _sc3_body(con_hbm, row_hbm, zs_hbm, agg_out,
              idxr0, idxr1, cb0, cb1, sb, semi0, semi1, agg_sp):
    cid = lax.axis_index("c")
    sid = lax.axis_index("s")
    w = sid * NC + cid
    base = sid * RPS
    nch = jnp.where(w < (NCHUNK % NW), NCHUNK // NW + 1, NCHUNK // NW)
    npairs = nch // 2

    pltpu.sync_copy(zs_hbm, sb)
    pltpu.sync_copy(sb, agg_sp.at[pl.ds(base, RPS), :])
    plsc.subcore_barrier()

    def off(t):
        return (t * NW + w) * CH

    def issue(t, idxr, cb, semi):
        pltpu.async_copy(row_hbm.at[pl.ds(off(t), CH)], idxr, semi)
        pltpu.async_copy(con_hbm.at[pl.ds(off(t), CH), :], cb, semi)

    def wait(idxr, cb, semi):
        pltpu.make_async_copy(row_hbm.at[pl.ds(0, CH)], idxr, semi).wait()
        pltpu.make_async_copy(con_hbm.at[pl.ds(0, CH), :], cb, semi).wait()

    issue(0, idxr0, cb0, semi0)
    issue(1, idxr1, cb1, semi1)

    def pair(u, carry):
        t0 = u * 2
        wait(idxr0, cb0, semi0)
        pltpu.sync_copy(cb0, agg_sp.at[idxr0], add=True)

        @pl.when(t0 + 2 < nch)
        def _():
            issue(t0 + 2, idxr0, cb0, semi0)

        wait(idxr1, cb1, semi1)
        pltpu.sync_copy(cb1, agg_sp.at[idxr1], add=True)

        @pl.when(t0 + 3 < nch)
        def _():
            issue(t0 + 3, idxr1, cb1, semi1)

        return carry

    lax.fori_loop(0, npairs, pair, 0)

    @pl.when(nch % 2 == 1)
    def _():
        wait(idxr0, cb0, semi0)
        pltpu.sync_copy(cb0, agg_sp.at[idxr0], add=True)

    plsc.subcore_barrier()

    pltpu.sync_copy(agg_sp.at[pl.ds(base, RPS), :], sb)
    pltpu.sync_copy(sb, agg_out.at[cid, pl.ds(base, RPS), :])


def _sc3(contrib, row, zs):
    f = functools.partial(
        pl.kernel, mesh=_mk_mesh(),
        compiler_params=pltpu.CompilerParams(use_tc_tiling_on_sc=False, needs_layout_passes=False),
        out_type=jax.ShapeDtypeStruct((NC, N, 16), jnp.float32),
        scratch_types=[
            pltpu.VMEM((CH,), jnp.int32), pltpu.VMEM((CH,), jnp.int32),
            pltpu.VMEM((CH, 16), jnp.float32), pltpu.VMEM((CH, 16), jnp.float32),
            pltpu.VMEM((RPS, 16), jnp.float32),
        ] + [pltpu.SemaphoreType.DMA] * 2 + [
            pltpu.VMEM_SHARED((N, 16), jnp.float32),
        ])(_sc3_body)
    return f(contrib, row, zs)


def _ln(x, eps=1e-6):
    m = jnp.mean(x, axis=-1, keepdims=True)
    v = jnp.mean((x - m) * (x - m), axis=-1, keepdims=True)
    return (x - m) / jnp.sqrt(v + eps)


def _silu(x):
    return x * (0.5 * jnp.tanh(0.5 * x) + 0.5)


def _mod(x, sh, sc):
    return x * (1 + sc) + sh


def _dot(a, b):
    return jax.lax.dot_general(a, b, (((1,), (0,)), ((), ())),
                               preferred_element_type=jnp.float32)


def _bcast_spec(arr):
    nd = arr.ndim
    return pl.BlockSpec(arr.shape, lambda i: (0,) * nd)


def _rows_spec(blk, cols):
    return pl.BlockSpec((blk, cols), lambda i: (i, 0))


_TC_PARAMS = pltpu.CompilerParams(dimension_semantics=("arbitrary",))


def _tcab_body(pdiff, ea, ete, ex8, h, nte,
               wd, we2, be2, wtm, btm, wl, wx8, ntm, nbtm, wq, wk, wv,
               misc_o, e1_o, e2_o, e3_o, e4_o,
               q_o, k_o, v_o, g1_o, g2_o, g3_o, g4_o):
    pd = pdiff[...]
    dist = jnp.sum(pd * pd, axis=-1, keepdims=True)
    e2 = dist * wd[...] + _dot(ea[...], we2[...]) + be2[...]
    tm = _dot(_silu(ete[...]), wtm[...]) + btm[...]
    emod = _mod(_ln(e2), tm[:, 0:16], tm[:, 16:32])
    lb = _dot(emod, wl[...]) + _dot(ex8[...], wx8[...])
    blk = lb.shape[0]
    misc_o[...] = jnp.concatenate(
        [lb, dist, jnp.zeros((blk, 7), jnp.float32)], axis=1)
    e1_o[...] = tm[:, 32:48]
    e2_o[...] = tm[:, 48:64]
    e3_o[...] = tm[:, 64:80]
    e4_o[...] = tm[:, 80:96]

    @pl.when(pl.program_id(0) < 10)
    def _():
        tmn = _dot(_silu(nte[...]), ntm[...]) + nbtm[...]
        hm = _mod(_ln(h[...]), tmn[:, 0:D], tmn[:, D:2 * D])
        q_o[...] = _dot(hm, wq[...])
        k_o[...] = _dot(hm, wk[...])
        v_o[...] = _dot(hm, wv[...])
        g1_o[...] = tmn[:, 2 * D:3 * D]
        g2_o[...] = tmn[:, 3 * D:4 * D]
        g3_o[...] = tmn[:, 4 * D:5 * D]
        g4_o[...] = tmn[:, 5 * D:6 * D]


def _tcab(pdiff, ea, ete, ex8, h, nte, p):
    blk = 2000
    nblk = 1000
    wd = p['edge_emb_W'][0:1]
    we2 = p['edge_emb_W'][1:]
    be2 = p['edge_emb_b'].reshape(1, -1)
    wtm, btm = p['edge_tm_W'], p['edge_tm_b'].reshape(1, -1)
    wx8 = jnp.pad(p['Wx'], ((0, 6), (0, 0)))
    ntm, nbtm = p['node_tm_W'], p['node_tm_b'].reshape(1, -1)

    def _nspec(cols):
        return pl.BlockSpec((nblk, cols), lambda i: (jnp.minimum(i, 9), 0))

    outs = ([jax.ShapeDtypeStruct((E, 16), jnp.float32)] * 5
            + [jax.ShapeDtypeStruct((N, D), jnp.float32)] * 7)
    return pl.pallas_call(
        _tcab_body,
        grid=(E // blk,),
        in_specs=[_rows_spec(blk, 16), _rows_spec(blk, 16),
                  _rows_spec(blk, D), _rows_spec(blk, 8),
                  _nspec(D), _nspec(D),
                  _bcast_spec(wd), _bcast_spec(we2), _bcast_spec(be2),
                  _bcast_spec(wtm), _bcast_spec(btm),
                  _bcast_spec(p['We']), _bcast_spec(wx8),
                  _bcast_spec(ntm), _bcast_spec(nbtm),
                  _bcast_spec(p['Wq']), _bcast_spec(p['Wk']), _bcast_spec(p['Wv'])],
        out_specs=[_rows_spec(blk, 16)] * 5 + [_nspec(D)] * 7,
        out_shape=outs,
        compiler_params=_TC_PARAMS,
    )(pdiff, ea, ete, ex8, h, nte, wd, we2, be2, wtm, btm, p['We'], wx8,
      ntm, nbtm, p['Wq'], p['Wk'], p['Wv'])


def _tcc_body(acc3, s3, h, nmask, g1, g2, g3, g4, sexp, wo, n2e,
              ff1, fb1, ff2, fb2, eqw1, eqw2,
              hout_o, b_o, c1_o, c2_o):
    acc = acc3[0] + acc3[1]
    sv = (s3[0] + s3[1])[:, 0:8]
    se = jax.lax.dot_general(sv, sexp[...], (((1,), (0,)), ((), ())),
                             precision=jax.lax.Precision.HIGHEST,
                             preferred_element_type=jnp.float32)
    att = _dot(acc / (se + 1e-16), wo[...])
    b_o[...] = _dot(att, n2e[...])
    h_node = h[...] + g1[...] * att
    hml = _mod(_ln(h_node), g2[...], g3[...]) * nmask[...]
    ffn = _dot(_silu(_dot(hml, ff1[...]) + fb1[...]), ff2[...]) + fb2[...]
    h_out = (hml + g4[...] * ffn) * nmask[...]
    hout_o[...] = h_out
    c1_o[...] = _dot(h_out, eqw1[...])
    c2_o[...] = _dot(h_out, eqw2[...])


def _tcc(acc_p, s_p, h, nmask, g1, g2, g3, g4, p):
    blk = 1000
    sexp = jnp.kron(jnp.eye(8, dtype=jnp.float32),
                    jnp.ones((1, DH), jnp.float32))
    fb1 = p['ff1_b'].reshape(1, -1)
    fb2 = p['ff2_b'].reshape(1, -1)
    eqw1 = p['eq_in_W'][0:D]
    eqw2 = p['eq_in_W'][D:2 * D]
    outs = [jax.ShapeDtypeStruct((N, D), jnp.float32),
            jax.ShapeDtypeStruct((N, 16), jnp.float32),
            jax.ShapeDtypeStruct((N, D), jnp.float32),
            jax.ShapeDtypeStruct((N, D), jnp.float32)]
    return pl.pallas_call(
        _tcc_body,
        grid=(N // blk,),
        in_specs=[pl.BlockSpec((NC, blk, D), lambda i: (0, i, 0)),
                  pl.BlockSpec((NC, blk, 16), lambda i: (0, i, 0)),
                  _rows_spec(blk, D), _rows_spec(blk, 1),
                  _rows_spec(blk, D), _rows_spec(blk, D),
                  _rows_spec(blk, D), _rows_spec(blk, D),
                  _bcast_spec(sexp), _bcast_spec(p['Wo']), _bcast_spec(p['n2e_W']),
                  _bcast_spec(p['ff1_W']), _bcast_spec(fb1),
                  _bcast_spec(p['ff2_W']), _bcast_spec(fb2),
                  _bcast_spec(eqw1), _bcast_spec(eqw2)],
        out_specs=[_rows_spec(blk, D), _rows_spec(blk, 16),
                   _rows_spec(blk, D), _rows_spec(blk, D)],
        out_shape=outs,
        compiler_params=_TC_PARAMS,
    )(acc_p, s_p, h, nmask, g1, g2, g3, g4, sexp, p['Wo'], p['n2e_W'],
      p['ff1_W'], fb1, p['ff2_W'], fb2, eqw1, eqw2)


def _tcde_body(ea, g1, g2, g3, g4, bsum, g1g, g2g, misc, pdiff, ete, ex8,
               n2eb, ff3, fb3, ff4, fb4, wtm, btm, w3, w4, eqb,
               c1w, c1b, c2w, cscale,
               heo_o, con_o):
    he = ea[...] + g1[...] * (bsum[...] + n2eb[...])
    he = _mod(_ln(he), g2[...], g3[...])
    ffe = _dot(_silu(_dot(he, ff3[...]) + fb3[...]), ff4[...]) + fb4[...]
    heo = he + g4[...] * ffe
    heo_o[...] = heo
    tm = _dot(_silu(ete[...]), wtm[...]) + btm[...]
    dist = misc[:, 8:9]
    lin = g1g[...] + g2g[...] + _dot(heo, w3[...]) + dist * w4[...] + eqb[...]
    inv = _mod(_ln(lin), tm[:, 0:D], tm[:, D:2 * D])
    u = jnp.tanh(_dot(_silu(_dot(inv, c1w[...]) + c1b[...]), c2w[...]))
    blk = u.shape[0]
    adjs = jnp.concatenate([jnp.ones((blk, 1), jnp.float32), ex8[:, 0:7]],
                           axis=1)
    invm = jnp.sum(u * adjs, axis=-1, keepdims=True) * (1.0 / 3.0)
    nrm = jnp.sqrt(dist)
    cdf = pdiff[...] / jnp.maximum(nrm, 1e-8) * cscale[...]
    con_o[...] = cdf * invm


def _tcde(ea, g1, g2, g3, g4, bsum, g1g, g2g, misc, pdiff, ete, ex8, p):
    blk = 2000
    n2eb = p['n2e_b'].reshape(1, -1)
    fb3 = p['ff3_b'].reshape(1, -1)
    fb4 = p['ff4_b'].reshape(1, -1)
    wtm, btm = p['eq_tm_W'], p['eq_tm_b'].reshape(1, -1)
    w3 = p['eq_in_W'][2 * D:2 * D + 16]
    w4 = p['eq_in_W'][2 * D + 16:2 * D + 17]
    eqb = p['eq_in_b'].reshape(1, -1)
    c1b = p['eq_c1_b'].reshape(1, -1)
    c2w = jnp.pad(p['eq_c2_W'], ((0, 0), (0, 5)))
    cscale = p['coors_scale'].reshape(1, 1)
    outs = [jax.ShapeDtypeStruct((E, 16), jnp.float32),
            jax.ShapeDtypeStruct((E, 16), jnp.float32)]
    return pl.pallas_call(
        _tcde_body,
        grid=(E // blk,),
        in_specs=[_rows_spec(blk, 16), _rows_spec(blk, 16), _rows_spec(blk, 16),
                  _rows_spec(blk, 16), _rows_spec(blk, 16), _rows_spec(blk, 16),
                  _rows_spec(blk, D), _rows_spec(blk, D), _rows_spec(blk, 16),
                  _rows_spec(blk, 16), _rows_spec(blk, D), _rows_spec(blk, 8),
                  _bcast_spec(n2eb), _bcast_spec(p['ff3_W']), _bcast_spec(fb3),
                  _bcast_spec(p['ff4_W']), _bcast_spec(fb4),
                  _bcast_spec(wtm), _bcast_spec(btm), _bcast_spec(w3),
                  _bcast_spec(w4), _bcast_spec(eqb),
                  _bcast_spec(p['eq_c1_W']), _bcast_spec(c1b), _bcast_spec(c2w),
                  _bcast_spec(cscale)],
        out_specs=[_rows_spec(blk, 16), _rows_spec(blk, 16)],
        out_shape=outs,
        compiler_params=_TC_PARAMS,
    )(ea, g1, g2, g3, g4, bsum, g1g, g2g, misc, pdiff, ete, ex8,
      n2eb, p['ff3_W'], fb3, p['ff4_W'], fb4, wtm, btm, w3, w4, eqb,
      p['eq_c1_W'], c1b, c2w, cscale)


def _tcf_body(pos16, agg3, out_o):
    out_o[...] = pos16[...] + agg3[0] + agg3[1]


def _tcf(pos16, agg_p):
    blk = 1000
    return pl.pallas_call(
        _tcf_body,
        grid=(N // blk,),
        in_specs=[_rows_spec(blk, 16),
                  pl.BlockSpec((NC, blk, 16), lambda i: (0, i, 0))],
        out_specs=_rows_spec(blk, 16),
        out_shape=jax.ShapeDtypeStruct((N, 16), jnp.float32),
        compiler_params=_TC_PARAMS,
    )(pos16, agg_p)


def kernel(pos, h, edge_attr, edge_index, node_mask, extra_heads,
           node_time_emb, edge_time_emb, params):
    p = params
    row = edge_index[0]
    col = edge_index[1]
    pos16 = jnp.pad(pos, ((0, 0), (0, 13)))
    ex8 = jnp.pad(extra_heads, ((0, 0), (0, 6)))
    zb = jnp.zeros((CH1, D), jnp.float32)
    zs1 = jnp.zeros((CH1, 16), jnp.float32)
    zs3 = jnp.zeros((RPS, 16), jnp.float32)

    pdiff = _sc0(pos16, row, col)
    (misc, e1, e2m, e3, e4,
     q, k, v, g1, g2, g3, g4) = _tcab(pdiff, edge_attr, edge_time_emb, ex8,
                                      h, node_time_emb, p)
    acc_p, s_p = _sc1(q, k, v, row, col, misc, zb, zs1)
    h_out, b_tab, c1, c2 = _tcc(acc_p, s_p, h, node_mask, g1, g2, g3, g4, p)
    bsum, g1g, g2g = _sc2(b_tab, c1, c2, row, col)
    h_edge_out, contrib = _tcde(edge_attr, e1, e2m, e3, e4, bsum, g1g, g2g,
                                misc, pdiff, edge_time_emb, ex8, p)
    agg_p = _sc3(contrib, row, zs3)
    pos_out16 = _tcf(pos16, agg_p)
    return h_out, h_edge_out, pos_out16[:, 0:3]
